# Initial kernel scaffold; baseline (speedup 1.0000x reference)
import jax
import jax.numpy as jnp
from jax.experimental import pallas as pl


def kernel(x, edge_index, edge_attr, global_attr, sp_L_values, coeff, num_processing_steps, emb, Wenc, benc, Web, beb, Wnb, bnb, Wgb, bgb, Wd1, bd1, Wd2, bd2, Wi1, bi1, Wi2, bi2):
    raise NotImplementedError("write your pallas kernel here")


# speedup vs baseline: 4.0904x; 4.0904x over previous
import functools

import jax
import jax.numpy as jnp
from jax import lax
from jax.experimental import pallas as pl
from jax.experimental.pallas import tpu as pltpu
from jax.experimental.pallas import tpu_sc as plsc

F32 = jnp.float32
I32 = jnp.int32


def _edge_pass_a(E, N, H, with_hep, write_enew):
    info = plsc.get_sparse_core_info()
    NC, NS, L = info.num_cores, info.num_subcores, info.num_lanes
    NW = NC * NS
    assert E % NW == 0
    EW = E // NW
    C = 128
    nfull, tail = divmod(EW, C)
    RW = -(-(N // NS) // 8) * 8
    JH = H // L

    mesh = plsc.VectorSubcoreMesh(core_axis_name="c", subcore_axis_name="s")

    outs = []
    if write_enew:
        outs.append(jax.ShapeDtypeStruct((E, H), F32))
    outs.append(jax.ShapeDtypeStruct((NC, N, H), F32))
    outs.append(jax.ShapeDtypeStruct((NC, N, H), F32))

    scratch = [
        pltpu.VMEM((C,), I32),
        pltpu.VMEM((C,), I32),
        pltpu.VMEM((C + L,), I32),
        pltpu.VMEM((C, H), F32),
        pltpu.VMEM((C, H), F32),
        pltpu.VMEM((C, H), F32),
        pltpu.VMEM((C, H), F32),
    ]
    if tail:
        scratch += [
            pltpu.VMEM((tail,), I32),
            pltpu.VMEM((tail,), I32),
            pltpu.VMEM((tail + L,), I32),
            pltpu.VMEM((tail, H), F32),
            pltpu.VMEM((tail, H), F32),
            pltpu.VMEM((tail, H), F32),
            pltpu.VMEM((tail, H), F32),
        ]
    scratch += [
        pltpu.VMEM((16, H), F32),
        pltpu.VMEM_SHARED((N, H), F32),
        pltpu.VMEM_SHARED((N, H), F32),
        pltpu.SemaphoreType.DMA,
        pltpu.SemaphoreType.DMA,
    ]

    @functools.partial(pl.kernel, mesh=mesh, out_type=tuple(outs),
                       scratch_types=scratch,
                       compiler_params=pltpu.CompilerParams(
                           use_tc_tiling_on_sc=False))
    def k(*refs):
        idx = 0
        src_h, dst_h, attr_h, s_h, d_h, t_h = refs[0:6]
        idx = 6
        if with_hep:
            hep_h = refs[idx]; idx += 1
        zeros_h = refs[idx]; idx += 1
        if write_enew:
            enew_h = refs[idx]; idx += 1
        racc_h = refs[idx]; idx += 1
        sacc_h = refs[idx]; idx += 1
        i_s, i_d, i_t, r_s, r_d, r_h, e_v = refs[idx:idx + 7]
        idx += 7
        if tail:
            ti_s, ti_d, ti_t, tr_s, tr_d, tr_h, te_v = refs[idx:idx + 7]
            idx += 7
        tab_v, racc_sh, sacc_sh, sem0, sem1 = refs[idx:idx + 5]

        cid = lax.axis_index("c")
        sid = lax.axis_index("s")
        wid = cid * NS + sid

        rstart = pl.multiple_of(jnp.minimum(sid * RW, N - RW), 8)
        zsl = pl.ds(rstart, RW)
        pltpu.sync_copy(zeros_h.at[zsl], racc_sh.at[zsl])
        pltpu.sync_copy(zeros_h.at[zsl], sacc_sh.at[zsl])
        pltpu.sync_copy(t_h, tab_v)
        plsc.subcore_barrier()

        base0 = wid * EW

        def chunk(base, cc, ci_s, ci_d, ci_t, cr_s, cr_d, cr_h, ce_v):
            sl = pl.ds(pl.multiple_of(base, 8), cc)
            pltpu.sync_copy(src_h.at[sl], ci_s)
            pltpu.sync_copy(dst_h.at[sl], ci_d)
            pltpu.sync_copy(attr_h.at[sl], ci_t.at[pl.ds(0, cc)])
            cs = pltpu.async_copy(s_h.at[ci_s], cr_s, sem0)
            cd = pltpu.async_copy(d_h.at[ci_d], cr_d, sem1)
            if with_hep:
                pltpu.sync_copy(hep_h.at[sl], cr_h)
            cs.wait()
            cd.wait()

            def row(i, _):
                a = ci_t[pl.ds(i, L)][0]
                for j in range(JH):
                    jl = pl.ds(j * L, L)
                    v = cr_s[i, jl] + cr_d[i, jl] + tab_v[a, jl]
                    if with_hep:
                        v = v + cr_h[i, jl]
                    ce_v[i, jl] = jnp.maximum(v, 0.0)
                return 0

            lax.fori_loop(0, cc, row, 0)
            pltpu.sync_copy(ce_v, racc_sh.at[ci_d], add=True)
            pltpu.sync_copy(ce_v, sacc_sh.at[ci_s], add=True)
            if write_enew:
                pltpu.sync_copy(ce_v, enew_h.at[sl])

        def body(kk, _):
            chunk(base0 + kk * C, C, i_s, i_d, i_t, r_s, r_d, r_h, e_v)
            return 0

        lax.fori_loop(0, nfull, body, 0)
        if tail:
            chunk(base0 + nfull * C, tail, ti_s, ti_d, ti_t, tr_s, tr_d,
                  tr_h, te_v)

        plsc.subcore_barrier()
        osl = pl.ds(rstart, RW)
        pltpu.sync_copy(racc_sh.at[osl], racc_h.at[cid, osl])
        pltpu.sync_copy(sacc_sh.at[osl], sacc_h.at[cid, osl])

    return k


def _edge_pass_b(E, N, H):
    info = plsc.get_sparse_core_info()
    NC, NS, L = info.num_cores, info.num_subcores, info.num_lanes
    NW = NC * NS
    EW = E // NW
    C = 128
    nfull, tail = divmod(EW, C)
    RW = -(-(N // NS) // 8) * 8
    JH = H // L

    mesh = plsc.VectorSubcoreMesh(core_axis_name="c", subcore_axis_name="s")

    scratch = [
        pltpu.VMEM((C,), I32),
        pltpu.VMEM((C,), I32),
        pltpu.VMEM((C + L,), F32),
        pltpu.VMEM((C, H), F32),
    ]
    if tail:
        scratch += [
            pltpu.VMEM((tail,), I32),
            pltpu.VMEM((tail,), I32),
            pltpu.VMEM((tail + L,), F32),
            pltpu.VMEM((tail, H), F32),
        ]
    scratch += [
        pltpu.VMEM((L,), F32),
        pltpu.VMEM_SHARED((N, H), F32),
        pltpu.SemaphoreType.DMA,
    ]

    @functools.partial(
        pl.kernel, mesh=mesh,
        out_type=jax.ShapeDtypeStruct((NC, N, H), F32),
        scratch_types=scratch,
        compiler_params=pltpu.CompilerParams(use_tc_tiling_on_sc=False))
    def k(*refs):
        src_h, dst_h, spl_h, coeff_h, nn_h, zeros_h, sd_h = refs[0:7]
        idx = 7
        i_s, i_d, v_l, r_n = refs[idx:idx + 4]
        idx += 4
        if tail:
            ti_s, ti_d, tv_l, tr_n = refs[idx:idx + 4]
            idx += 4
        c_v, sd_sh, sem0 = refs[idx:idx + 3]

        cid = lax.axis_index("c")
        sid = lax.axis_index("s")
        wid = cid * NS + sid

        rstart = pl.multiple_of(jnp.minimum(sid * RW, N - RW), 8)
        zsl = pl.ds(rstart, RW)
        pltpu.sync_copy(zeros_h.at[zsl], sd_sh.at[zsl])
        pltpu.sync_copy(coeff_h, c_v)
        plsc.subcore_barrier()

        coef = c_v[...][0]
        base0 = wid * EW

        def chunk(base, cc, ci_s, ci_d, cv_l, cr_n):
            sl = pl.ds(pl.multiple_of(base, 8), cc)
            pltpu.sync_copy(src_h.at[sl], ci_s)
            pltpu.sync_copy(dst_h.at[sl], ci_d)
            pltpu.sync_copy(spl_h.at[sl], cv_l.at[pl.ds(0, cc)])
            pltpu.async_copy(nn_h.at[ci_s], cr_n, sem0).wait()

            def row(i, _):
                s = cv_l[pl.ds(i, L)][0] * coef
                for j in range(JH):
                    jl = pl.ds(j * L, L)
                    cr_n[i, jl] = cr_n[i, jl] * s
                return 0

            lax.fori_loop(0, cc, row, 0)
            pltpu.sync_copy(cr_n, sd_sh.at[ci_d], add=True)

        def body(kk, _):
            chunk(base0 + kk * C, C, i_s, i_d, v_l, r_n)
            return 0

        lax.fori_loop(0, nfull, body, 0)
        if tail:
            chunk(base0 + nfull * C, tail, ti_s, ti_d, tv_l, tr_n)

        plsc.subcore_barrier()
        osl = pl.ds(rstart, RW)
        pltpu.sync_copy(sd_sh.at[osl], sd_h.at[cid, osl])

    return k


def _dot(a, b):
    return jnp.dot(a, b, preferred_element_type=F32)


def _k1_prep(N, D, H, K, BN):
    ng = N // BN

    def body(x0, x1, wenc, benc, wixs, wixd, emb, wie, wg, beb, g0, wng, bnb,
             ix0, ix1, s0, d0, embw, table0, nbias0):
        a0 = jnp.maximum(_dot(x0[...], wenc[...]) + benc[...], 0.0)
        a1 = jnp.maximum(_dot(x1[...], wenc[...]) + benc[...], 0.0)
        ix0[...] = a0
        ix1[...] = a1
        s0[...] = _dot(a0, wixs[...])
        d0[...] = _dot(a0, wixd[...])
        ew = _dot(emb[...], wie[...])
        embw[...] = ew
        gv = _dot(g0[...], wg[...]) + beb[...]
        table0[...] = ew + gv
        nbias0[...] = bnb[...] + _dot(g0[...], wng[...])

    full = lambda i: (0, 0)
    blk = lambda i: (i, 0)
    return pl.pallas_call(
        body,
        grid=(ng,),
        in_specs=[
            pl.BlockSpec((BN, D), blk), pl.BlockSpec((BN, D), blk),
            pl.BlockSpec((D, H), full), pl.BlockSpec((1, H), full),
            pl.BlockSpec((H, H), full), pl.BlockSpec((H, H), full),
            pl.BlockSpec((K, H), full), pl.BlockSpec((H, H), full),
            pl.BlockSpec((H, H), full), pl.BlockSpec((1, H), full),
            pl.BlockSpec((1, H), full), pl.BlockSpec((H, H), full),
            pl.BlockSpec((1, H), full),
        ],
        out_specs=[
            pl.BlockSpec((BN, H), blk), pl.BlockSpec((BN, H), blk),
            pl.BlockSpec((BN, H), blk), pl.BlockSpec((BN, H), blk),
            pl.BlockSpec((K, H), full), pl.BlockSpec((K, H), full),
            pl.BlockSpec((1, H), full),
        ],
        out_shape=[
            jax.ShapeDtypeStruct((N, H), F32), jax.ShapeDtypeStruct((N, H), F32),
            jax.ShapeDtypeStruct((N, H), F32), jax.ShapeDtypeStruct((N, H), F32),
            jax.ShapeDtypeStruct((K, H), F32), jax.ShapeDtypeStruct((K, H), F32),
            jax.ShapeDtypeStruct((1, H), F32),
        ],
    )


def _k2_node0(N, E, H, K, NC, BN):
    ng = N // BN

    def body(ix0, ix1, racc, sacc, wnix, wnrecv, wnsent, nbias0,
             whxs, whxd, wixs, wixd, embw, wg, beb, wgb, bgb, wng, bnb, g0,
             n0, s1, d1, table1, nbias1, nsum, rsum):
        i = pl.program_id(0)
        recv = racc[0] + racc[1]
        sent = sacc[0] + sacc[1]
        a0 = jnp.maximum(
            _dot(ix0[...], wnix[...]) + _dot(recv, wnrecv[...])
            + _dot(sent, wnsent[...]) + nbias0[...], 0.0)
        n0[...] = a0
        s1[...] = _dot(a0, whxs[...]) + _dot(ix1[...], wixs[...])
        d1[...] = _dot(a0, whxd[...]) + _dot(ix1[...], wixd[...])

        @pl.when(i == 0)
        def _():
            nsum[...] = jnp.zeros_like(nsum)
            rsum[...] = jnp.zeros_like(rsum)

        nsum[...] += jnp.sum(a0, axis=0, keepdims=True)
        rsum[...] += jnp.sum(recv, axis=0, keepdims=True)

        @pl.when(i == ng - 1)
        def _():
            n_mean = nsum[...] / float(N)
            e_mean = rsum[...] / float(E)
            g_in = jnp.concatenate([n_mean, e_mean, g0[...]], axis=1)
            g1 = jnp.maximum(_dot(g_in, wgb[...]) + bgb[...], 0.0)
            table1[...] = embw[...] + _dot(g1, wg[...]) + beb[...]
            nbias1[...] = bnb[...] + _dot(g1, wng[...])

    full = lambda i: (0, 0)
    blk = lambda i: (i, 0)
    blk3 = lambda i: (0, i, 0)
    return pl.pallas_call(
        body,
        grid=(ng,),
        in_specs=[
            pl.BlockSpec((BN, H), blk), pl.BlockSpec((BN, H), blk),
            pl.BlockSpec((NC, BN, H), blk3), pl.BlockSpec((NC, BN, H), blk3),
            pl.BlockSpec((H, H), full), pl.BlockSpec((H, H), full),
            pl.BlockSpec((H, H), full), pl.BlockSpec((1, H), full),
            pl.BlockSpec((H, H), full), pl.BlockSpec((H, H), full),
            pl.BlockSpec((H, H), full), pl.BlockSpec((H, H), full),
            pl.BlockSpec((K, H), full), pl.BlockSpec((H, H), full),
            pl.BlockSpec((1, H), full), pl.BlockSpec((3 * H, H), full),
            pl.BlockSpec((1, H), full), pl.BlockSpec((H, H), full),
            pl.BlockSpec((1, H), full), pl.BlockSpec((1, H), full),
        ],
        out_specs=[
            pl.BlockSpec((BN, H), blk), pl.BlockSpec((BN, H), blk),
            pl.BlockSpec((BN, H), blk), pl.BlockSpec((K, H), full),
            pl.BlockSpec((1, H), full), pl.BlockSpec((1, H), full),
            pl.BlockSpec((1, H), full),
        ],
        out_shape=[
            jax.ShapeDtypeStruct((N, H), F32), jax.ShapeDtypeStruct((N, H), F32),
            jax.ShapeDtypeStruct((N, H), F32), jax.ShapeDtypeStruct((K, H), F32),
            jax.ShapeDtypeStruct((1, H), F32), jax.ShapeDtypeStruct((1, H), F32),
            jax.ShapeDtypeStruct((1, H), F32),
        ],
    )


def _k3_heproj(E, H, BE):
    ng = E // BE

    def body(e0, whe, out):
        out[...] = _dot(e0[...], whe[...])

    return pl.pallas_call(
        body,
        grid=(ng,),
        in_specs=[pl.BlockSpec((BE, H), lambda i: (i, 0)),
                  pl.BlockSpec((H, H), lambda i: (0, 0))],
        out_specs=pl.BlockSpec((BE, H), lambda i: (i, 0)),
        out_shape=jax.ShapeDtypeStruct((E, H), F32),
    )


def _k4_node1(N, H, NC, BN):
    ng = N // BN

    def body(n0, ix1, racc, sacc, wnhx, wnix, wnrecv, wnsent, nbias1, sdacc0,
             n1, td1, sd0):
        recv = racc[0] + racc[1]
        sent = sacc[0] + sacc[1]
        a1 = jnp.maximum(
            _dot(n0[...], wnhx[...]) + _dot(ix1[...], wnix[...])
            + _dot(recv, wnrecv[...]) + _dot(sent, wnsent[...])
            + nbias1[...], 0.0)
        n1[...] = a1
        td1[...] = a1 - n0[...]
        sd0[...] = sdacc0[0] + sdacc0[1]

    full = lambda i: (0, 0)
    blk = lambda i: (i, 0)
    blk3 = lambda i: (0, i, 0)
    return pl.pallas_call(
        body,
        grid=(ng,),
        in_specs=[
            pl.BlockSpec((BN, H), blk), pl.BlockSpec((BN, H), blk),
            pl.BlockSpec((NC, BN, H), blk3), pl.BlockSpec((NC, BN, H), blk3),
            pl.BlockSpec((H, H), full), pl.BlockSpec((H, H), full),
            pl.BlockSpec((H, H), full), pl.BlockSpec((H, H), full),
            pl.BlockSpec((1, H), full), pl.BlockSpec((NC, BN, H), blk3),
        ],
        out_specs=[pl.BlockSpec((BN, H), blk), pl.BlockSpec((BN, H), blk),
                   pl.BlockSpec((BN, H), blk)],
        out_shape=[jax.ShapeDtypeStruct((N, H), F32),
                   jax.ShapeDtypeStruct((N, H), F32),
                   jax.ShapeDtypeStruct((N, H), F32)],
    )


def _k5_dec(N, H, D, NC, BN):
    ng = N // BN

    def body(n0, n1, sdacc1, wd1, bd1, wd2p, bd2p, wi1, bi1, wi2, bi2,
             o0, o1, p0, p1, sd1):
        h00 = jnp.maximum(_dot(n0[...], wd1[...]) + bd1[...], 0.0)
        h01 = jnp.maximum(_dot(n1[...], wd1[...]) + bd1[...], 0.0)
        o0[...] = _dot(h00, wd2p[...]) + bd2p[...]
        o1[...] = _dot(h01, wd2p[...]) + bd2p[...]
        h10 = jnp.maximum(_dot(n0[...], wi1[...]) + bi1[...], 0.0)
        h11 = jnp.maximum(_dot(n1[...], wi1[...]) + bi1[...], 0.0)
        p0[...] = _dot(h10, wi2[...]) + bi2[...]
        p1[...] = _dot(h11, wi2[...]) + bi2[...]
        sd1[...] = sdacc1[0] + sdacc1[1]

    full = lambda i: (0, 0)
    blk = lambda i: (i, 0)
    blk3 = lambda i: (0, i, 0)
    return pl.pallas_call(
        body,
        grid=(ng,),
        in_specs=[
            pl.BlockSpec((BN, H), blk), pl.BlockSpec((BN, H), blk),
            pl.BlockSpec((NC, BN, H), blk3),
            pl.BlockSpec((H, H), full), pl.BlockSpec((1, H), full),
            pl.BlockSpec((H, D), full), pl.BlockSpec((1, D), full),
            pl.BlockSpec((H, H), full), pl.BlockSpec((1, H), full),
            pl.BlockSpec((H, D), full), pl.BlockSpec((1, D), full),
        ],
        out_specs=[pl.BlockSpec((BN, D), blk), pl.BlockSpec((BN, D), blk),
                   pl.BlockSpec((BN, D), blk), pl.BlockSpec((BN, D), blk),
                   pl.BlockSpec((BN, H), blk)],
        out_shape=[jax.ShapeDtypeStruct((N, D), F32),
                   jax.ShapeDtypeStruct((N, D), F32),
                   jax.ShapeDtypeStruct((N, D), F32),
                   jax.ShapeDtypeStruct((N, D), F32),
                   jax.ShapeDtypeStruct((N, H), F32)],
    )


def kernel(x, edge_index, edge_attr, global_attr, sp_L_values, coeff,
           num_processing_steps, emb, Wenc, benc, Web, beb, Wnb, bnb, Wgb,
           bgb, Wd1, bd1, Wd2, bd2, Wi1, bi1, Wi2, bi2):
    T, N, D = x.shape
    E = edge_index.shape[1]
    H = Wenc.shape[1]
    K = emb.shape[0]
    OUT = Wd2.shape[1]
    assert T == 2

    info = plsc.get_sparse_core_info()
    NC = info.num_cores
    BN = 1000
    BE = 2000

    src = edge_index[0]
    dst = edge_index[1]
    attr0 = edge_attr[0]
    attr1 = edge_attr[1]

    W_he, W_hxs, W_hxd, W_ie, W_ixs, W_ixd, W_g = (
        Web[i * H:(i + 1) * H] for i in range(7))
    Wn_hx, Wn_ix, Wn_recv, Wn_sent, Wn_g = (
        Wnb[i * H:(i + 1) * H] for i in range(5))

    g0 = global_attr
    r = lambda v: v.reshape(1, -1)
    zeros_nh = jnp.zeros((N, H), F32)
    coeff16 = jnp.concatenate([coeff, jnp.zeros((15,), F32)])
    Wd2p = jnp.pad(Wd2, ((0, 0), (0, D - OUT)))
    bd2p = jnp.pad(bd2, (0, D - OUT)).reshape(1, D)

    ix0, ix1, S0, D0, embW, table0, nbias0 = _k1_prep(N, D, H, K, BN)(
        x[0], x[1], Wenc, r(benc), W_ixs, W_ixd, emb, W_ie, W_g, r(beb),
        g0, Wn_g, r(bnb))

    e0, racc0, sacc0 = _edge_pass_a(E, N, H, with_hep=False, write_enew=True)(
        src, dst, attr0, S0, D0, table0, zeros_nh)

    n0, S1, D1, table1, nbias1, _, _ = _k2_node0(N, E, H, K, NC, BN)(
        ix0, ix1, racc0, sacc0, Wn_ix, Wn_recv, Wn_sent, nbias0,
        W_hxs, W_hxd, W_ixs, W_ixd, embW, W_g, r(beb), Wgb, r(bgb),
        Wn_g, r(bnb), g0)

    sdacc0 = _edge_pass_b(E, N, H)(src, dst, sp_L_values, coeff16, n0,
                                   zeros_nh)

    hep1 = _k3_heproj(E, H, BE)(e0, W_he)

    racc1, sacc1 = _edge_pass_a(E, N, H, with_hep=True, write_enew=False)(
        src, dst, attr1, S1, D1, table1, hep1, zeros_nh)

    n1, td1, sd0 = _k4_node1(N, H, NC, BN)(
        n0, ix1, racc1, sacc1, Wn_hx, Wn_ix, Wn_recv, Wn_sent, nbias1,
        sdacc0)

    sdacc1 = _edge_pass_b(E, N, H)(src, dst, sp_L_values, coeff16, n1,
                                   zeros_nh)

    o0, o1, p0, p1, sd1 = _k5_dec(N, H, D, NC, BN)(
        n0, n1, sdacc1, Wd1, r(bd1), Wd2p, bd2p, Wi1, r(bi1), Wi2, r(bi2))

    out_nodes = jnp.stack([o0[:, :OUT], o1[:, :OUT]])
    time_derivatives = jnp.stack([n0, td1])
    spatial_derivatives = jnp.stack([sd0, sd1])
    pred_inputs = jnp.stack([p0, p1])
    return (out_nodes, time_derivatives, spatial_derivatives, pred_inputs)


# speedup vs baseline: 4.3818x; 1.0712x over previous
import functools

import jax
import jax.numpy as jnp
from jax import lax
from jax.experimental import pallas as pl
from jax.experimental.pallas import tpu as pltpu
from jax.experimental.pallas import tpu_sc as plsc

F32 = jnp.float32
I32 = jnp.int32


def _edge_pass_a(E, N, H, write_enew):
    info = plsc.get_sparse_core_info()
    NC, NS, L = info.num_cores, info.num_subcores, info.num_lanes
    NW = NC * NS
    assert E % NW == 0
    EW = E // NW
    C = 64
    nfull, tail = divmod(EW, C)
    assert nfull % 3 == 0 and nfull >= 6
    RW = -(-(N // NS) // 8) * 8
    JH = H // L

    mesh = plsc.VectorSubcoreMesh(core_axis_name="c", subcore_axis_name="s")

    outs = []
    if write_enew:
        outs.append(jax.ShapeDtypeStruct((E, H), F32))
    outs.append(jax.ShapeDtypeStruct((NC, N, H), F32))
    outs.append(jax.ShapeDtypeStruct((NC, N, H), F32))

    def bufset(cc):
        return [
            pltpu.VMEM((2, cc), I32),
            pltpu.VMEM((cc, H), F32),
            pltpu.VMEM((cc, H), F32),
            pltpu.VMEM((cc, H), F32),
        ]

    scratch = []
    for _ in range(3):
        scratch += bufset(C)
    if tail:
        scratch += bufset(tail)
    scratch += [
        pltpu.VMEM_SHARED((N, H), F32),
        pltpu.VMEM_SHARED((N, H), F32),
    ]
    scratch += [pltpu.SemaphoreType.DMA] * 6

    @functools.partial(pl.kernel, mesh=mesh, out_type=tuple(outs),
                       scratch_types=scratch,
                       compiler_params=pltpu.CompilerParams(
                           use_tc_tiling_on_sc=False))
    def k(*refs):
        ei_h, s_h, d_h, hep_h, zeros_h = refs[0:5]
        idx = 5
        if write_enew:
            enew_h = refs[idx]; idx += 1
        racc_h = refs[idx]; idx += 1
        sacc_h = refs[idx]; idx += 1
        sets = [refs[idx + 4 * t: idx + 4 * (t + 1)] for t in range(3)]
        idx += 12
        if tail:
            tset = refs[idx:idx + 4]
            idx += 4
        racc_sh, sacc_sh = refs[idx:idx + 2]
        idx += 2
        gsem = refs[idx:idx + 3]
        ssem = refs[idx + 3:idx + 6]

        cid = lax.axis_index("c")
        sid = lax.axis_index("s")
        wid = cid * NS + sid

        rstart = pl.multiple_of(jnp.minimum(sid * RW, N - RW), 8)
        zsl = pl.ds(rstart, RW)
        pltpu.sync_copy(zeros_h.at[zsl], racc_sh.at[zsl])
        pltpu.sync_copy(zeros_h.at[zsl], sacc_sh.at[zsl])
        plsc.subcore_barrier()

        base0 = wid * EW

        def cbase(kk):
            return pl.multiple_of(
                jnp.minimum(base0 + kk * C, E - C), 8)

        def fetch(kk, t):
            ci, cr_s, cr_d, cr_h = sets[t]
            sl = pl.ds(cbase(kk), C)
            pltpu.sync_copy(ei_h.at[:, sl], ci)
            hs = pltpu.async_copy(s_h.at[ci.at[0]], cr_s, gsem[t])
            hd = pltpu.async_copy(d_h.at[ci.at[1]], cr_d, gsem[t])
            hh = pltpu.async_copy(hep_h.at[sl], cr_h, gsem[t])
            return (hs, hd, hh)

        def compute(t):
            ci, cr_s, cr_d, cr_h = sets[t]

            def row(i, _):
                for j in range(JH):
                    jl = pl.ds(j * L, L)
                    v = cr_s[i, jl] + cr_d[i, jl] + cr_h[i, jl]
                    cr_s[i, jl] = jnp.maximum(v, 0.0)
                return 0

            lax.fori_loop(0, C, row, 0)

        def scatter(kk, t):
            ci, ce_v, _, _ = sets[t]
            pltpu.sync_copy(ce_v, racc_sh.at[ci.at[1]], add=True)
            pltpu.sync_copy(ce_v, sacc_sh.at[ci.at[0]], add=True)
            if write_enew:
                pltpu.sync_copy(ce_v, enew_h.at[pl.ds(cbase(kk), C)])

        def group(k0):
            hs = [fetch(k0 + j, j) for j in range(3)]
            for j in range(3):
                for h in hs[j]:
                    h.wait()
                compute(j)
                scatter(k0 + j, j)

        def body(m, _):
            group(3 * m)
            return 0

        lax.fori_loop(0, nfull // 3, body, 0)

        if tail:
            tci, tr_s, tr_d, tr_h = tset
            te_v = tr_s
            sl = pl.ds(pl.multiple_of(base0 + nfull * C, 8), tail)
            pltpu.sync_copy(ei_h.at[:, sl], tci)
            ths = pltpu.async_copy(s_h.at[tci.at[0]], tr_s, gsem[0])
            thd = pltpu.async_copy(d_h.at[tci.at[1]], tr_d, gsem[0])
            pltpu.sync_copy(hep_h.at[sl], tr_h)
            ths.wait()
            thd.wait()

            def trow(i, _):
                for j in range(JH):
                    jl = pl.ds(j * L, L)
                    v = tr_s[i, jl] + tr_d[i, jl] + tr_h[i, jl]
                    te_v[i, jl] = jnp.maximum(v, 0.0)
                return 0

            lax.fori_loop(0, tail, trow, 0)
            pltpu.sync_copy(te_v, racc_sh.at[tci.at[1]], add=True)
            pltpu.sync_copy(te_v, sacc_sh.at[tci.at[0]], add=True)
            if write_enew:
                pltpu.sync_copy(te_v, enew_h.at[sl])

        plsc.subcore_barrier()
        osl = pl.ds(rstart, RW)
        pltpu.sync_copy(racc_sh.at[osl], racc_h.at[cid, osl])
        pltpu.sync_copy(sacc_sh.at[osl], sacc_h.at[cid, osl])

    return k


def _edge_pass_b(E, N, H):
    info = plsc.get_sparse_core_info()
    NC, NS, L = info.num_cores, info.num_subcores, info.num_lanes
    NW = NC * NS
    EW = E // NW
    C = 64
    nfull, tail = divmod(EW, C)
    assert nfull % 3 == 0 and nfull >= 6
    RW = -(-(N // NS) // 8) * 8
    JH = H // L

    mesh = plsc.VectorSubcoreMesh(core_axis_name="c", subcore_axis_name="s")

    def bufset(cc):
        return [
            pltpu.VMEM((2, cc), I32),
            pltpu.VMEM((cc + L,), F32),
            pltpu.VMEM((cc, H), F32),
        ]

    scratch = []
    for _ in range(3):
        scratch += bufset(C)
    if tail:
        scratch += bufset(tail)
    scratch += [
        pltpu.VMEM((L,), F32),
        pltpu.VMEM_SHARED((N, H), F32),
    ]
    scratch += [pltpu.SemaphoreType.DMA] * 6

    @functools.partial(
        pl.kernel, mesh=mesh,
        out_type=jax.ShapeDtypeStruct((NC, N, H), F32),
        scratch_types=scratch,
        compiler_params=pltpu.CompilerParams(use_tc_tiling_on_sc=False))
    def k(*refs):
        ei_h, spl_h, coeff_h, nn_h, zeros_h, sd_h = refs[0:6]
        idx = 6
        sets = [refs[idx + 3 * t: idx + 3 * (t + 1)] for t in range(3)]
        idx += 9
        if tail:
            tset = refs[idx:idx + 3]
            idx += 3
        c_v, sd_sh = refs[idx:idx + 2]
        idx += 2
        gsem = refs[idx:idx + 3]
        ssem = refs[idx + 3:idx + 6]

        cid = lax.axis_index("c")
        sid = lax.axis_index("s")
        wid = cid * NS + sid

        rstart = pl.multiple_of(jnp.minimum(sid * RW, N - RW), 8)
        zsl = pl.ds(rstart, RW)
        pltpu.sync_copy(zeros_h.at[zsl], sd_sh.at[zsl])
        pltpu.sync_copy(coeff_h, c_v)
        plsc.subcore_barrier()

        coef = c_v[...][0]
        base0 = wid * EW

        def cbase(kk):
            return pl.multiple_of(
                jnp.minimum(base0 + kk * C, E - C), 8)

        def fetch(kk, t):
            ci, cv_l, cr_n = sets[t]
            sl = pl.ds(cbase(kk), C)
            pltpu.sync_copy(ei_h.at[:, sl], ci)
            pltpu.sync_copy(spl_h.at[sl], cv_l.at[pl.ds(0, C)])
            return pltpu.async_copy(nn_h.at[ci.at[0]], cr_n, gsem[t])

        def compute(t):
            ci, cv_l, cr_n = sets[t]

            def row(i, _):
                s = cv_l[pl.ds(i, L)][0] * coef
                for j in range(JH):
                    jl = pl.ds(j * L, L)
                    cr_n[i, jl] = cr_n[i, jl] * s
                return 0

            lax.fori_loop(0, C, row, 0)

        def scatter(t):
            ci, cv_l, cr_n = sets[t]
            pltpu.sync_copy(cr_n, sd_sh.at[ci.at[1]], add=True)

        def group(k0):
            hs = [fetch(k0 + j, j) for j in range(3)]
            for j in range(3):
                hs[j].wait()
                compute(j)
                scatter(j)

        def body(m, _):
            group(3 * m)
            return 0

        lax.fori_loop(0, nfull // 3, body, 0)

        if tail:
            tci, tv_l, tr_n = tset
            sl = pl.ds(pl.multiple_of(base0 + nfull * C, 8), tail)
            pltpu.sync_copy(ei_h.at[:, sl], tci)
            pltpu.sync_copy(spl_h.at[sl], tv_l.at[pl.ds(0, tail)])
            pltpu.async_copy(nn_h.at[tci.at[0]], tr_n, gsem[0]).wait()

            def trow(i, _):
                s = tv_l[pl.ds(i, L)][0] * coef
                for j in range(JH):
                    jl = pl.ds(j * L, L)
                    tr_n[i, jl] = tr_n[i, jl] * s
                return 0

            lax.fori_loop(0, tail, trow, 0)
            pltpu.sync_copy(tr_n, sd_sh.at[tci.at[1]], add=True)

        plsc.subcore_barrier()
        osl = pl.ds(rstart, RW)
        pltpu.sync_copy(sd_sh.at[osl], sd_h.at[cid, osl])

    return k


def _dot(a, b):
    return jnp.dot(a, b, preferred_element_type=F32,
                   precision=lax.Precision.HIGHEST)


def _k1_prep(N, D, H, K, BN):
    ng = N // BN

    def body(x0, x1, wenc, benc, wixs, wixd, emb, wie, wg, beb, g0, wng, bnb,
             ix0, ix1, s0, d0, embw, table0, nbias0):
        a0 = jnp.maximum(_dot(x0[...], wenc[...]) + benc[...], 0.0)
        a1 = jnp.maximum(_dot(x1[...], wenc[...]) + benc[...], 0.0)
        ix0[...] = a0
        ix1[...] = a1
        s0[...] = _dot(a0, wixs[...])
        d0[...] = _dot(a0, wixd[...])
        ew = _dot(emb[...], wie[...])
        embw[...] = ew
        gv = _dot(g0[...], wg[...]) + beb[...]
        table0[...] = ew + gv
        nbias0[...] = bnb[...] + _dot(g0[...], wng[...])

    full = lambda i: (0, 0)
    blk = lambda i: (i, 0)
    return pl.pallas_call(
        body,
        grid=(ng,),
        in_specs=[
            pl.BlockSpec((BN, D), blk), pl.BlockSpec((BN, D), blk),
            pl.BlockSpec((D, H), full), pl.BlockSpec((1, H), full),
            pl.BlockSpec((H, H), full), pl.BlockSpec((H, H), full),
            pl.BlockSpec((K, H), full), pl.BlockSpec((H, H), full),
            pl.BlockSpec((H, H), full), pl.BlockSpec((1, H), full),
            pl.BlockSpec((1, H), full), pl.BlockSpec((H, H), full),
            pl.BlockSpec((1, H), full),
        ],
        out_specs=[
            pl.BlockSpec((BN, H), blk), pl.BlockSpec((BN, H), blk),
            pl.BlockSpec((BN, H), blk), pl.BlockSpec((BN, H), blk),
            pl.BlockSpec((K, H), full), pl.BlockSpec((K, H), full),
            pl.BlockSpec((1, H), full),
        ],
        out_shape=[
            jax.ShapeDtypeStruct((N, H), F32), jax.ShapeDtypeStruct((N, H), F32),
            jax.ShapeDtypeStruct((N, H), F32), jax.ShapeDtypeStruct((N, H), F32),
            jax.ShapeDtypeStruct((K, H), F32), jax.ShapeDtypeStruct((K, H), F32),
            jax.ShapeDtypeStruct((1, H), F32),
        ],
    )


def _k2_node0(N, E, H, K, NC, BN):
    ng = N // BN

    def body(ix0, ix1, racc, sacc, wnix, wnrecv, wnsent, nbias0,
             whxs, whxd, wixs, wixd, embw, wg, beb, wgb, bgb, wng, bnb, g0,
             n0, sn1, d1, table1, nbias1, nsum, rsum):
        i = pl.program_id(0)
        recv = racc[0] + racc[1]
        sent = sacc[0] + sacc[1]
        a0 = jnp.maximum(
            _dot(ix0[...], wnix[...]) + _dot(recv, wnrecv[...])
            + _dot(sent, wnsent[...]) + nbias0[...], 0.0)
        n0[...] = a0
        sn1[...] = _dot(a0, whxs[...]) + _dot(ix1[...], wixs[...])
        d1[...] = _dot(a0, whxd[...]) + _dot(ix1[...], wixd[...])

        @pl.when(i == 0)
        def _():
            nsum[...] = jnp.zeros_like(nsum)
            rsum[...] = jnp.zeros_like(rsum)

        nsum[...] += jnp.sum(a0, axis=0, keepdims=True)
        rsum[...] += jnp.sum(recv, axis=0, keepdims=True)

        @pl.when(i == ng - 1)
        def _():
            n_mean = nsum[...] / float(N)
            e_mean = rsum[...] / float(E)
            g_in = jnp.concatenate([n_mean, e_mean, g0[...]], axis=1)
            g1 = jnp.maximum(_dot(g_in, wgb[...]) + bgb[...], 0.0)
            table1[...] = embw[...] + _dot(g1, wg[...]) + beb[...]
            nbias1[...] = bnb[...] + _dot(g1, wng[...])

    full = lambda i: (0, 0)
    blk = lambda i: (i, 0)
    blk3 = lambda i: (0, i, 0)
    return pl.pallas_call(
        body,
        grid=(ng,),
        in_specs=[
            pl.BlockSpec((BN, H), blk), pl.BlockSpec((BN, H), blk),
            pl.BlockSpec((NC, BN, H), blk3), pl.BlockSpec((NC, BN, H), blk3),
            pl.BlockSpec((H, H), full), pl.BlockSpec((H, H), full),
            pl.BlockSpec((H, H), full), pl.BlockSpec((1, H), full),
            pl.BlockSpec((H, H), full), pl.BlockSpec((H, H), full),
            pl.BlockSpec((H, H), full), pl.BlockSpec((H, H), full),
            pl.BlockSpec((K, H), full), pl.BlockSpec((H, H), full),
            pl.BlockSpec((1, H), full), pl.BlockSpec((3 * H, H), full),
            pl.BlockSpec((1, H), full), pl.BlockSpec((H, H), full),
            pl.BlockSpec((1, H), full), pl.BlockSpec((1, H), full),
        ],
        out_specs=[
            pl.BlockSpec((BN, H), blk), pl.BlockSpec((BN, H), blk),
            pl.BlockSpec((BN, H), blk), pl.BlockSpec((K, H), full),
            pl.BlockSpec((1, H), full), pl.BlockSpec((1, H), full),
            pl.BlockSpec((1, H), full),
        ],
        out_shape=[
            jax.ShapeDtypeStruct((N, H), F32),
            jax.ShapeDtypeStruct((N, H), F32),
            jax.ShapeDtypeStruct((N, H), F32), jax.ShapeDtypeStruct((K, H), F32),
            jax.ShapeDtypeStruct((1, H), F32), jax.ShapeDtypeStruct((1, H), F32),
            jax.ShapeDtypeStruct((1, H), F32),
        ],
    )


def _k3_heproj(E, H, K, BE, with_he):
    ng = E // BE

    def body(*refs):
        if with_he:
            attr3, tab, e0, whe, out = refs
        else:
            attr3, tab, out = refs
        a = attr3[0]
        kio = lax.broadcasted_iota(I32, (K, BE), 0).astype(F32)
        oht = (kio == jnp.broadcast_to(a, (K, BE))).astype(F32)
        tie = lax.dot_general(oht, tab[...], (((0,), (0,)), ((), ())),
                              preferred_element_type=F32,
                              precision=lax.Precision.HIGHEST)
        if with_he:
            tie = tie + _dot(e0[...], whe[...])
        out[...] = tie

    in_specs = [pl.BlockSpec((1, 1, BE), lambda i: (i, 0, 0)),
                pl.BlockSpec((K, H), lambda i: (0, 0))]
    if with_he:
        in_specs += [pl.BlockSpec((BE, H), lambda i: (i, 0)),
                     pl.BlockSpec((H, H), lambda i: (0, 0))]
    return pl.pallas_call(
        body,
        grid=(ng,),
        in_specs=in_specs,
        out_specs=pl.BlockSpec((BE, H), lambda i: (i, 0)),
        out_shape=jax.ShapeDtypeStruct((E, H), F32),
    )


def _k4_node1(N, H, NC, BN):
    ng = N // BN

    def body(n0, ix1, racc, sacc, wnhx, wnix, wnrecv, wnsent, nbias1, sdacc0,
             n1, td1, sd0):
        recv = racc[0] + racc[1]
        sent = sacc[0] + sacc[1]
        a1 = jnp.maximum(
            _dot(n0[...], wnhx[...]) + _dot(ix1[...], wnix[...])
            + _dot(recv, wnrecv[...]) + _dot(sent, wnsent[...])
            + nbias1[...], 0.0)
        n1[...] = a1
        td1[...] = a1 - n0[...]
        sd0[...] = sdacc0[0] + sdacc0[1]

    full = lambda i: (0, 0)
    blk = lambda i: (i, 0)
    blk3 = lambda i: (0, i, 0)
    return pl.pallas_call(
        body,
        grid=(ng,),
        in_specs=[
            pl.BlockSpec((BN, H), blk), pl.BlockSpec((BN, H), blk),
            pl.BlockSpec((NC, BN, H), blk3), pl.BlockSpec((NC, BN, H), blk3),
            pl.BlockSpec((H, H), full), pl.BlockSpec((H, H), full),
            pl.BlockSpec((H, H), full), pl.BlockSpec((H, H), full),
            pl.BlockSpec((1, H), full), pl.BlockSpec((NC, BN, H), blk3),
        ],
        out_specs=[pl.BlockSpec((BN, H), blk), pl.BlockSpec((BN, H), blk),
                   pl.BlockSpec((BN, H), blk)],
        out_shape=[jax.ShapeDtypeStruct((N, H), F32),
                   jax.ShapeDtypeStruct((N, H), F32),
                   jax.ShapeDtypeStruct((N, H), F32)],
    )


def _k5_dec(N, H, D, NC, BN):
    ng = N // BN

    def body(n0, n1, sdacc1, wd1, bd1, wd2p, bd2p, wi1, bi1, wi2, bi2,
             o0, o1, p0, p1, sd1):
        h00 = jnp.maximum(_dot(n0[...], wd1[...]) + bd1[...], 0.0)
        h01 = jnp.maximum(_dot(n1[...], wd1[...]) + bd1[...], 0.0)
        o0[...] = _dot(h00, wd2p[...]) + bd2p[...]
        o1[...] = _dot(h01, wd2p[...]) + bd2p[...]
        h10 = jnp.maximum(_dot(n0[...], wi1[...]) + bi1[...], 0.0)
        h11 = jnp.maximum(_dot(n1[...], wi1[...]) + bi1[...], 0.0)
        p0[...] = _dot(h10, wi2[...]) + bi2[...]
        p1[...] = _dot(h11, wi2[...]) + bi2[...]
        sd1[...] = sdacc1[0] + sdacc1[1]

    full = lambda i: (0, 0)
    blk = lambda i: (i, 0)
    blk3 = lambda i: (0, i, 0)
    return pl.pallas_call(
        body,
        grid=(ng,),
        in_specs=[
            pl.BlockSpec((BN, H), blk), pl.BlockSpec((BN, H), blk),
            pl.BlockSpec((NC, BN, H), blk3),
            pl.BlockSpec((H, H), full), pl.BlockSpec((1, H), full),
            pl.BlockSpec((H, D), full), pl.BlockSpec((1, D), full),
            pl.BlockSpec((H, H), full), pl.BlockSpec((1, H), full),
            pl.BlockSpec((H, D), full), pl.BlockSpec((1, D), full),
        ],
        out_specs=[pl.BlockSpec((BN, D), blk), pl.BlockSpec((BN, D), blk),
                   pl.BlockSpec((BN, D), blk), pl.BlockSpec((BN, D), blk),
                   pl.BlockSpec((BN, H), blk)],
        out_shape=[jax.ShapeDtypeStruct((N, D), F32),
                   jax.ShapeDtypeStruct((N, D), F32),
                   jax.ShapeDtypeStruct((N, D), F32),
                   jax.ShapeDtypeStruct((N, D), F32),
                   jax.ShapeDtypeStruct((N, H), F32)],
    )


def kernel(x, edge_index, edge_attr, global_attr, sp_L_values, coeff,
           num_processing_steps, emb, Wenc, benc, Web, beb, Wnb, bnb, Wgb,
           bgb, Wd1, bd1, Wd2, bd2, Wi1, bi1, Wi2, bi2):
    T, N, D = x.shape
    E = edge_index.shape[1]
    H = Wenc.shape[1]
    K = emb.shape[0]
    OUT = Wd2.shape[1]
    assert T == 2

    info = plsc.get_sparse_core_info()
    NC = info.num_cores
    BN = 1000

    attrf = edge_attr.astype(F32)
    BE = 2000
    attr0f = attrf[0].reshape(E // BE, 1, BE)
    attr1f = attrf[1].reshape(E // BE, 1, BE)

    W_he, W_hxs, W_hxd, W_ie, W_ixs, W_ixd, W_g = (
        Web[i * H:(i + 1) * H] for i in range(7))
    Wn_hx, Wn_ix, Wn_recv, Wn_sent, Wn_g = (
        Wnb[i * H:(i + 1) * H] for i in range(5))

    g0 = global_attr
    r = lambda v: v.reshape(1, -1)
    zeros_nh = jnp.zeros((N, H), F32)
    coeff16 = jnp.concatenate([coeff, jnp.zeros((15,), F32)])
    Wd2p = jnp.pad(Wd2, ((0, 0), (0, D - OUT)))
    bd2p = jnp.pad(bd2, (0, D - OUT)).reshape(1, D)

    ix0, ix1, S0, D0, embW, table0, nbias0 = _k1_prep(N, D, H, K, BN)(
        x[0], x[1], Wenc, r(benc), W_ixs, W_ixd, emb, W_ie, W_g, r(beb),
        g0, Wn_g, r(bnb))

    tie0 = _k3_heproj(E, H, K, BE, with_he=False)(attr0f, table0)

    e0, racc0, sacc0 = _edge_pass_a(E, N, H, write_enew=True)(
        edge_index, S0, D0, tie0, zeros_nh)

    n0, S1, D1, table1, nbias1, _, _ = _k2_node0(N, E, H, K, NC, BN)(
        ix0, ix1, racc0, sacc0, Wn_ix, Wn_recv, Wn_sent, nbias0,
        W_hxs, W_hxd, W_ixs, W_ixd, embW, W_g, r(beb), Wgb, r(bgb),
        Wn_g, r(bnb), g0)

    sdacc0 = _edge_pass_b(E, N, H)(edge_index, sp_L_values, coeff16, n0,
                                   zeros_nh)

    hep1 = _k3_heproj(E, H, K, BE, with_he=True)(attr1f, table1, e0, W_he)

    racc1, sacc1 = _edge_pass_a(E, N, H, write_enew=False)(
        edge_index, S1, D1, hep1, zeros_nh)

    n1, td1, sd0 = _k4_node1(N, H, NC, BN)(
        n0, ix1, racc1, sacc1, Wn_hx, Wn_ix, Wn_recv, Wn_sent, nbias1,
        sdacc0)

    sdacc1 = _edge_pass_b(E, N, H)(edge_index, sp_L_values, coeff16, n1,
                                   zeros_nh)

    o0, o1, p0, p1, sd1 = _k5_dec(N, H, D, NC, BN)(
        n0, n1, sdacc1, Wd1, r(bd1), Wd2p, bd2p, Wi1, r(bi1), Wi2, r(bi2))

    out_nodes = jnp.stack([o0[:, :OUT], o1[:, :OUT]])
    time_derivatives = jnp.stack([n0, td1])
    spatial_derivatives = jnp.stack([sd0, sd1])
    pred_inputs = jnp.stack([p0, p1])
    return (out_nodes, time_derivatives, spatial_derivatives, pred_inputs)


# speedup vs baseline: 4.5000x; 1.0270x over previous
import functools

import jax
import jax.numpy as jnp
from jax import lax
from jax.experimental import pallas as pl
from jax.experimental.pallas import tpu as pltpu
from jax.experimental.pallas import tpu_sc as plsc

F32 = jnp.float32
I32 = jnp.int32


def _edge_pass_a(E, N, H, write_enew):
    info = plsc.get_sparse_core_info()
    NC, NS, L = info.num_cores, info.num_subcores, info.num_lanes
    NW = NC * NS
    assert E % NW == 0
    EW = E // NW
    C = 64
    nfull, tail = divmod(EW, C)
    assert nfull % 3 == 0 and nfull >= 6
    RW = -(-(N // NS) // 8) * 8
    JH = H // L

    mesh = plsc.VectorSubcoreMesh(core_axis_name="c", subcore_axis_name="s")

    outs = []
    if write_enew:
        outs.append(jax.ShapeDtypeStruct((E, H), F32))
    outs.append(jax.ShapeDtypeStruct((NC, N, H), F32))
    outs.append(jax.ShapeDtypeStruct((NC, N, H), F32))

    def bufset(cc):
        return [
            pltpu.VMEM((2, cc), I32),
            pltpu.VMEM((cc, H), F32),
            pltpu.VMEM((cc, H), F32),
            pltpu.VMEM((cc, H), F32),
        ]

    scratch = []
    for _ in range(3):
        scratch += bufset(C)
    if tail:
        scratch += bufset(tail)
    scratch += [
        pltpu.VMEM_SHARED((N, H), F32),
        pltpu.VMEM_SHARED((N, H), F32),
    ]
    scratch += [pltpu.SemaphoreType.DMA] * 9

    @functools.partial(pl.kernel, mesh=mesh, out_type=tuple(outs),
                       scratch_types=scratch,
                       compiler_params=pltpu.CompilerParams(
                           use_tc_tiling_on_sc=False))
    def k(*refs):
        ei_h, s_h, d_h, hep_h, zeros_h = refs[0:5]
        idx = 5
        if write_enew:
            enew_h = refs[idx]; idx += 1
        racc_h = refs[idx]; idx += 1
        sacc_h = refs[idx]; idx += 1
        sets = [refs[idx + 4 * t: idx + 4 * (t + 1)] for t in range(3)]
        idx += 12
        if tail:
            tset = refs[idx:idx + 4]
            idx += 4
        racc_sh, sacc_sh = refs[idx:idx + 2]
        idx += 2
        gsem = refs[idx:idx + 3]
        ssem = refs[idx + 3:idx + 6]
        isem = refs[idx + 6:idx + 9]

        cid = lax.axis_index("c")
        sid = lax.axis_index("s")
        wid = cid * NS + sid

        rstart = pl.multiple_of(jnp.minimum(sid * RW, N - RW), 8)
        zsl = pl.ds(rstart, RW)
        pltpu.sync_copy(zeros_h.at[zsl], racc_sh.at[zsl])
        pltpu.sync_copy(zeros_h.at[zsl], sacc_sh.at[zsl])
        plsc.subcore_barrier()

        base0 = wid * EW

        def cbase(kk):
            return pl.multiple_of(
                jnp.minimum(base0 + kk * C, E - C), 8)

        def fetch_idx(kk, t):
            ci = sets[t][0]
            sl = pl.ds(cbase(kk), C)
            return pltpu.async_copy(ei_h.at[:, sl], ci, isem[t])

        def fetch(kk, t):
            ci, cr_s, cr_d, cr_h = sets[t]
            sl = pl.ds(cbase(kk), C)
            hs = pltpu.async_copy(s_h.at[ci.at[0]], cr_s, gsem[t])
            hd = pltpu.async_copy(d_h.at[ci.at[1]], cr_d, gsem[t])
            hh = pltpu.async_copy(hep_h.at[sl], cr_h, gsem[t])
            return (hs, hd, hh)

        def compute(t):
            ci, cr_s, cr_d, cr_h = sets[t]

            def row(i, _):
                for j in range(JH):
                    jl = pl.ds(j * L, L)
                    v = cr_s[i, jl] + cr_d[i, jl] + cr_h[i, jl]
                    cr_s[i, jl] = jnp.maximum(v, 0.0)
                return 0

            lax.fori_loop(0, C, row, 0)

        def scatter(kk, t):
            ci, ce_v, _, _ = sets[t]
            out = []
            if write_enew:
                out.append(pltpu.async_copy(
                    ce_v, enew_h.at[pl.ds(cbase(kk), C)], ssem[t]))
            pltpu.sync_copy(ce_v, racc_sh.at[ci.at[1]], add=True)
            pltpu.sync_copy(ce_v, sacc_sh.at[ci.at[0]], add=True)
            return out

        def group(k0):
            ihs = [fetch_idx(k0 + j, j) for j in range(3)]
            ghs = []
            for j in range(3):
                ihs[j].wait()
                ghs.append(fetch(k0 + j, j))
            shs = []
            for j in range(3):
                for h in ghs[j]:
                    h.wait()
                compute(j)
                shs += scatter(k0 + j, j)
            for h in shs:
                h.wait()

        def body(m, _):
            group(3 * m)
            return 0

        lax.fori_loop(0, nfull // 3, body, 0)

        if tail:
            tci, tr_s, tr_d, tr_h = tset
            te_v = tr_s
            sl = pl.ds(pl.multiple_of(base0 + nfull * C, 8), tail)
            pltpu.sync_copy(ei_h.at[:, sl], tci)
            ths = pltpu.async_copy(s_h.at[tci.at[0]], tr_s, gsem[0])
            thd = pltpu.async_copy(d_h.at[tci.at[1]], tr_d, gsem[0])
            pltpu.sync_copy(hep_h.at[sl], tr_h)
            ths.wait()
            thd.wait()

            def trow(i, _):
                for j in range(JH):
                    jl = pl.ds(j * L, L)
                    v = tr_s[i, jl] + tr_d[i, jl] + tr_h[i, jl]
                    te_v[i, jl] = jnp.maximum(v, 0.0)
                return 0

            lax.fori_loop(0, tail, trow, 0)
            pltpu.sync_copy(te_v, racc_sh.at[tci.at[1]], add=True)
            pltpu.sync_copy(te_v, sacc_sh.at[tci.at[0]], add=True)
            if write_enew:
                pltpu.sync_copy(te_v, enew_h.at[sl])

        plsc.subcore_barrier()
        osl = pl.ds(rstart, RW)
        pltpu.sync_copy(racc_sh.at[osl], racc_h.at[cid, osl])
        pltpu.sync_copy(sacc_sh.at[osl], sacc_h.at[cid, osl])

    return k


def _edge_pass_b(E, N, H):
    info = plsc.get_sparse_core_info()
    NC, NS, L = info.num_cores, info.num_subcores, info.num_lanes
    NW = NC * NS
    EW = E // NW
    C = 64
    nfull, tail = divmod(EW, C)
    assert nfull % 3 == 0 and nfull >= 6
    RW = -(-(N // NS) // 8) * 8
    JH = H // L

    mesh = plsc.VectorSubcoreMesh(core_axis_name="c", subcore_axis_name="s")

    def bufset(cc):
        return [
            pltpu.VMEM((2, cc), I32),
            pltpu.VMEM((cc + L,), F32),
            pltpu.VMEM((cc, H), F32),
        ]

    scratch = []
    for _ in range(3):
        scratch += bufset(C)
    if tail:
        scratch += bufset(tail)
    scratch += [
        pltpu.VMEM((L,), F32),
        pltpu.VMEM_SHARED((N, H), F32),
    ]
    scratch += [pltpu.SemaphoreType.DMA] * 9

    @functools.partial(
        pl.kernel, mesh=mesh,
        out_type=jax.ShapeDtypeStruct((NC, N, H), F32),
        scratch_types=scratch,
        compiler_params=pltpu.CompilerParams(use_tc_tiling_on_sc=False))
    def k(*refs):
        ei_h, spl_h, coeff_h, nn_h, zeros_h, sd_h = refs[0:6]
        idx = 6
        sets = [refs[idx + 3 * t: idx + 3 * (t + 1)] for t in range(3)]
        idx += 9
        if tail:
            tset = refs[idx:idx + 3]
            idx += 3
        c_v, sd_sh = refs[idx:idx + 2]
        idx += 2
        gsem = refs[idx:idx + 3]
        ssem = refs[idx + 3:idx + 6]
        isem = refs[idx + 6:idx + 9]

        cid = lax.axis_index("c")
        sid = lax.axis_index("s")
        wid = cid * NS + sid

        rstart = pl.multiple_of(jnp.minimum(sid * RW, N - RW), 8)
        zsl = pl.ds(rstart, RW)
        pltpu.sync_copy(zeros_h.at[zsl], sd_sh.at[zsl])
        pltpu.sync_copy(coeff_h, c_v)
        plsc.subcore_barrier()

        coef = c_v[...][0]
        base0 = wid * EW

        def cbase(kk):
            return pl.multiple_of(
                jnp.minimum(base0 + kk * C, E - C), 8)

        def fetch_idx(kk, t):
            ci, cv_l, cr_n = sets[t]
            sl = pl.ds(cbase(kk), C)
            return pltpu.async_copy(ei_h.at[:, sl], ci, isem[t])

        def fetch(kk, t):
            ci, cv_l, cr_n = sets[t]
            sl = pl.ds(cbase(kk), C)
            hl = pltpu.async_copy(spl_h.at[sl], cv_l.at[pl.ds(0, C)],
                                  gsem[t])
            hn = pltpu.async_copy(nn_h.at[ci.at[0]], cr_n, gsem[t])
            return (hl, hn)

        def compute(t):
            ci, cv_l, cr_n = sets[t]

            def row(i, _):
                s = cv_l[pl.ds(i, L)][0] * coef
                for j in range(JH):
                    jl = pl.ds(j * L, L)
                    cr_n[i, jl] = cr_n[i, jl] * s
                return 0

            lax.fori_loop(0, C, row, 0)

        def scatter(t):
            ci, cv_l, cr_n = sets[t]
            pltpu.sync_copy(cr_n, sd_sh.at[ci.at[1]], add=True)

        def group(k0):
            ihs = [fetch_idx(k0 + j, j) for j in range(3)]
            ghs = []
            for j in range(3):
                ihs[j].wait()
                ghs.append(fetch(k0 + j, j))
            for j in range(3):
                for h in ghs[j]:
                    h.wait()
                compute(j)
                scatter(j)

        def body(m, _):
            group(3 * m)
            return 0

        lax.fori_loop(0, nfull // 3, body, 0)

        if tail:
            tci, tv_l, tr_n = tset
            sl = pl.ds(pl.multiple_of(base0 + nfull * C, 8), tail)
            pltpu.sync_copy(ei_h.at[:, sl], tci)
            pltpu.sync_copy(spl_h.at[sl], tv_l.at[pl.ds(0, tail)])
            pltpu.async_copy(nn_h.at[tci.at[0]], tr_n, gsem[0]).wait()

            def trow(i, _):
                s = tv_l[pl.ds(i, L)][0] * coef
                for j in range(JH):
                    jl = pl.ds(j * L, L)
                    tr_n[i, jl] = tr_n[i, jl] * s
                return 0

            lax.fori_loop(0, tail, trow, 0)
            pltpu.sync_copy(tr_n, sd_sh.at[tci.at[1]], add=True)

        plsc.subcore_barrier()
        osl = pl.ds(rstart, RW)
        pltpu.sync_copy(sd_sh.at[osl], sd_h.at[cid, osl])

    return k


def _dot(a, b):
    return jnp.dot(a, b, preferred_element_type=F32,
                   precision=lax.Precision.HIGHEST)


def _k1_prep(N, D, H, K, BN):
    ng = N // BN

    def body(x0, x1, wenc, benc, wixs, wixd, emb, wie, wg, beb, g0, wng, bnb,
             ix0, ix1, s0, d0, embw, table0, nbias0):
        a0 = jnp.maximum(_dot(x0[...], wenc[...]) + benc[...], 0.0)
        a1 = jnp.maximum(_dot(x1[...], wenc[...]) + benc[...], 0.0)
        ix0[...] = a0
        ix1[...] = a1
        s0[...] = _dot(a0, wixs[...])
        d0[...] = _dot(a0, wixd[...])
        ew = _dot(emb[...], wie[...])
        embw[...] = ew
        gv = _dot(g0[...], wg[...]) + beb[...]
        table0[...] = ew + gv
        nbias0[...] = bnb[...] + _dot(g0[...], wng[...])

    full = lambda i: (0, 0)
    blk = lambda i: (i, 0)
    return pl.pallas_call(
        body,
        grid=(ng,),
        in_specs=[
            pl.BlockSpec((BN, D), blk), pl.BlockSpec((BN, D), blk),
            pl.BlockSpec((D, H), full), pl.BlockSpec((1, H), full),
            pl.BlockSpec((H, H), full), pl.BlockSpec((H, H), full),
            pl.BlockSpec((K, H), full), pl.BlockSpec((H, H), full),
            pl.BlockSpec((H, H), full), pl.BlockSpec((1, H), full),
            pl.BlockSpec((1, H), full), pl.BlockSpec((H, H), full),
            pl.BlockSpec((1, H), full),
        ],
        out_specs=[
            pl.BlockSpec((BN, H), blk), pl.BlockSpec((BN, H), blk),
            pl.BlockSpec((BN, H), blk), pl.BlockSpec((BN, H), blk),
            pl.BlockSpec((K, H), full), pl.BlockSpec((K, H), full),
            pl.BlockSpec((1, H), full),
        ],
        out_shape=[
            jax.ShapeDtypeStruct((N, H), F32), jax.ShapeDtypeStruct((N, H), F32),
            jax.ShapeDtypeStruct((N, H), F32), jax.ShapeDtypeStruct((N, H), F32),
            jax.ShapeDtypeStruct((K, H), F32), jax.ShapeDtypeStruct((K, H), F32),
            jax.ShapeDtypeStruct((1, H), F32),
        ],
    )


def _k2_node0(N, E, H, K, NC, BN):
    ng = N // BN

    def body(ix0, ix1, racc, sacc, wnix, wnrecv, wnsent, nbias0,
             whxs, whxd, wixs, wixd, embw, wg, beb, wgb, bgb, wng, bnb, g0,
             n0, sn1, d1, table1, nbias1, nsum, rsum):
        i = pl.program_id(0)
        recv = racc[0] + racc[1]
        sent = sacc[0] + sacc[1]
        a0 = jnp.maximum(
            _dot(ix0[...], wnix[...]) + _dot(recv, wnrecv[...])
            + _dot(sent, wnsent[...]) + nbias0[...], 0.0)
        n0[...] = a0
        sn1[...] = _dot(a0, whxs[...]) + _dot(ix1[...], wixs[...])
        d1[...] = _dot(a0, whxd[...]) + _dot(ix1[...], wixd[...])

        @pl.when(i == 0)
        def _():
            nsum[...] = jnp.zeros_like(nsum)
            rsum[...] = jnp.zeros_like(rsum)

        nsum[...] += jnp.sum(a0, axis=0, keepdims=True)
        rsum[...] += jnp.sum(recv, axis=0, keepdims=True)

        @pl.when(i == ng - 1)
        def _():
            n_mean = nsum[...] / float(N)
            e_mean = rsum[...] / float(E)
            g_in = jnp.concatenate([n_mean, e_mean, g0[...]], axis=1)
            g1 = jnp.maximum(_dot(g_in, wgb[...]) + bgb[...], 0.0)
            table1[...] = embw[...] + _dot(g1, wg[...]) + beb[...]
            nbias1[...] = bnb[...] + _dot(g1, wng[...])

    full = lambda i: (0, 0)
    blk = lambda i: (i, 0)
    blk3 = lambda i: (0, i, 0)
    return pl.pallas_call(
        body,
        grid=(ng,),
        in_specs=[
            pl.BlockSpec((BN, H), blk), pl.BlockSpec((BN, H), blk),
            pl.BlockSpec((NC, BN, H), blk3), pl.BlockSpec((NC, BN, H), blk3),
            pl.BlockSpec((H, H), full), pl.BlockSpec((H, H), full),
            pl.BlockSpec((H, H), full), pl.BlockSpec((1, H), full),
            pl.BlockSpec((H, H), full), pl.BlockSpec((H, H), full),
            pl.BlockSpec((H, H), full), pl.BlockSpec((H, H), full),
            pl.BlockSpec((K, H), full), pl.BlockSpec((H, H), full),
            pl.BlockSpec((1, H), full), pl.BlockSpec((3 * H, H), full),
            pl.BlockSpec((1, H), full), pl.BlockSpec((H, H), full),
            pl.BlockSpec((1, H), full), pl.BlockSpec((1, H), full),
        ],
        out_specs=[
            pl.BlockSpec((BN, H), blk), pl.BlockSpec((BN, H), blk),
            pl.BlockSpec((BN, H), blk), pl.BlockSpec((K, H), full),
            pl.BlockSpec((1, H), full), pl.BlockSpec((1, H), full),
            pl.BlockSpec((1, H), full),
        ],
        out_shape=[
            jax.ShapeDtypeStruct((N, H), F32),
            jax.ShapeDtypeStruct((N, H), F32),
            jax.ShapeDtypeStruct((N, H), F32), jax.ShapeDtypeStruct((K, H), F32),
            jax.ShapeDtypeStruct((1, H), F32), jax.ShapeDtypeStruct((1, H), F32),
            jax.ShapeDtypeStruct((1, H), F32),
        ],
    )


def _k3_heproj(E, H, K, BE, with_he):
    ng = E // BE

    def body(*refs):
        if with_he:
            attr3, tab, e0, whe, out = refs
        else:
            attr3, tab, out = refs
        a = attr3[0]
        kio = lax.broadcasted_iota(I32, (K, BE), 0).astype(F32)
        oht = (kio == jnp.broadcast_to(a, (K, BE))).astype(F32)
        tie = lax.dot_general(oht, tab[...], (((0,), (0,)), ((), ())),
                              preferred_element_type=F32,
                              precision=lax.Precision.HIGHEST)
        if with_he:
            tie = tie + _dot(e0[...], whe[...])
        out[...] = tie

    in_specs = [pl.BlockSpec((1, 1, BE), lambda i: (i, 0, 0)),
                pl.BlockSpec((K, H), lambda i: (0, 0))]
    if with_he:
        in_specs += [pl.BlockSpec((BE, H), lambda i: (i, 0)),
                     pl.BlockSpec((H, H), lambda i: (0, 0))]
    return pl.pallas_call(
        body,
        grid=(ng,),
        in_specs=in_specs,
        out_specs=pl.BlockSpec((BE, H), lambda i: (i, 0)),
        out_shape=jax.ShapeDtypeStruct((E, H), F32),
    )


def _k4_node1(N, H, NC, BN):
    ng = N // BN

    def body(n0, ix1, racc, sacc, wnhx, wnix, wnrecv, wnsent, nbias1, sdacc0,
             n1, td1, sd0):
        recv = racc[0] + racc[1]
        sent = sacc[0] + sacc[1]
        a1 = jnp.maximum(
            _dot(n0[...], wnhx[...]) + _dot(ix1[...], wnix[...])
            + _dot(recv, wnrecv[...]) + _dot(sent, wnsent[...])
            + nbias1[...], 0.0)
        n1[...] = a1
        td1[...] = a1 - n0[...]
        sd0[...] = sdacc0[0] + sdacc0[1]

    full = lambda i: (0, 0)
    blk = lambda i: (i, 0)
    blk3 = lambda i: (0, i, 0)
    return pl.pallas_call(
        body,
        grid=(ng,),
        in_specs=[
            pl.BlockSpec((BN, H), blk), pl.BlockSpec((BN, H), blk),
            pl.BlockSpec((NC, BN, H), blk3), pl.BlockSpec((NC, BN, H), blk3),
            pl.BlockSpec((H, H), full), pl.BlockSpec((H, H), full),
            pl.BlockSpec((H, H), full), pl.BlockSpec((H, H), full),
            pl.BlockSpec((1, H), full), pl.BlockSpec((NC, BN, H), blk3),
        ],
        out_specs=[pl.BlockSpec((BN, H), blk), pl.BlockSpec((BN, H), blk),
                   pl.BlockSpec((BN, H), blk)],
        out_shape=[jax.ShapeDtypeStruct((N, H), F32),
                   jax.ShapeDtypeStruct((N, H), F32),
                   jax.ShapeDtypeStruct((N, H), F32)],
    )


def _k5_dec(N, H, D, NC, BN):
    ng = N // BN

    def body(n0, n1, sdacc1, wd1, bd1, wd2p, bd2p, wi1, bi1, wi2, bi2,
             o0, o1, p0, p1, sd1):
        h00 = jnp.maximum(_dot(n0[...], wd1[...]) + bd1[...], 0.0)
        h01 = jnp.maximum(_dot(n1[...], wd1[...]) + bd1[...], 0.0)
        o0[...] = _dot(h00, wd2p[...]) + bd2p[...]
        o1[...] = _dot(h01, wd2p[...]) + bd2p[...]
        h10 = jnp.maximum(_dot(n0[...], wi1[...]) + bi1[...], 0.0)
        h11 = jnp.maximum(_dot(n1[...], wi1[...]) + bi1[...], 0.0)
        p0[...] = _dot(h10, wi2[...]) + bi2[...]
        p1[...] = _dot(h11, wi2[...]) + bi2[...]
        sd1[...] = sdacc1[0] + sdacc1[1]

    full = lambda i: (0, 0)
    blk = lambda i: (i, 0)
    blk3 = lambda i: (0, i, 0)
    return pl.pallas_call(
        body,
        grid=(ng,),
        in_specs=[
            pl.BlockSpec((BN, H), blk), pl.BlockSpec((BN, H), blk),
            pl.BlockSpec((NC, BN, H), blk3),
            pl.BlockSpec((H, H), full), pl.BlockSpec((1, H), full),
            pl.BlockSpec((H, D), full), pl.BlockSpec((1, D), full),
            pl.BlockSpec((H, H), full), pl.BlockSpec((1, H), full),
            pl.BlockSpec((H, D), full), pl.BlockSpec((1, D), full),
        ],
        out_specs=[pl.BlockSpec((BN, D), blk), pl.BlockSpec((BN, D), blk),
                   pl.BlockSpec((BN, D), blk), pl.BlockSpec((BN, D), blk),
                   pl.BlockSpec((BN, H), blk)],
        out_shape=[jax.ShapeDtypeStruct((N, D), F32),
                   jax.ShapeDtypeStruct((N, D), F32),
                   jax.ShapeDtypeStruct((N, D), F32),
                   jax.ShapeDtypeStruct((N, D), F32),
                   jax.ShapeDtypeStruct((N, H), F32)],
    )


def kernel(x, edge_index, edge_attr, global_attr, sp_L_values, coeff,
           num_processing_steps, emb, Wenc, benc, Web, beb, Wnb, bnb, Wgb,
           bgb, Wd1, bd1, Wd2, bd2, Wi1, bi1, Wi2, bi2):
    T, N, D = x.shape
    E = edge_index.shape[1]
    H = Wenc.shape[1]
    K = emb.shape[0]
    OUT = Wd2.shape[1]
    assert T == 2

    info = plsc.get_sparse_core_info()
    NC = info.num_cores
    BN = 1000

    attrf = edge_attr.astype(F32)
    BE = 2000
    attr0f = attrf[0].reshape(E // BE, 1, BE)
    attr1f = attrf[1].reshape(E // BE, 1, BE)

    W_he, W_hxs, W_hxd, W_ie, W_ixs, W_ixd, W_g = (
        Web[i * H:(i + 1) * H] for i in range(7))
    Wn_hx, Wn_ix, Wn_recv, Wn_sent, Wn_g = (
        Wnb[i * H:(i + 1) * H] for i in range(5))

    g0 = global_attr
    r = lambda v: v.reshape(1, -1)
    zeros_nh = jnp.zeros((N, H), F32)
    coeff16 = jnp.concatenate([coeff, jnp.zeros((15,), F32)])
    Wd2p = jnp.pad(Wd2, ((0, 0), (0, D - OUT)))
    bd2p = jnp.pad(bd2, (0, D - OUT)).reshape(1, D)

    ix0, ix1, S0, D0, embW, table0, nbias0 = _k1_prep(N, D, H, K, BN)(
        x[0], x[1], Wenc, r(benc), W_ixs, W_ixd, emb, W_ie, W_g, r(beb),
        g0, Wn_g, r(bnb))

    tie0 = _k3_heproj(E, H, K, BE, with_he=False)(attr0f, table0)

    e0, racc0, sacc0 = _edge_pass_a(E, N, H, write_enew=True)(
        edge_index, S0, D0, tie0, zeros_nh)

    n0, S1, D1, table1, nbias1, _, _ = _k2_node0(N, E, H, K, NC, BN)(
        ix0, ix1, racc0, sacc0, Wn_ix, Wn_recv, Wn_sent, nbias0,
        W_hxs, W_hxd, W_ixs, W_ixd, embW, W_g, r(beb), Wgb, r(bgb),
        Wn_g, r(bnb), g0)

    sdacc0 = _edge_pass_b(E, N, H)(edge_index, sp_L_values, coeff16, n0,
                                   zeros_nh)

    hep1 = _k3_heproj(E, H, K, BE, with_he=True)(attr1f, table1, e0, W_he)

    racc1, sacc1 = _edge_pass_a(E, N, H, write_enew=False)(
        edge_index, S1, D1, hep1, zeros_nh)

    n1, td1, sd0 = _k4_node1(N, H, NC, BN)(
        n0, ix1, racc1, sacc1, Wn_hx, Wn_ix, Wn_recv, Wn_sent, nbias1,
        sdacc0)

    sdacc1 = _edge_pass_b(E, N, H)(edge_index, sp_L_values, coeff16, n1,
                                   zeros_nh)

    o0, o1, p0, p1, sd1 = _k5_dec(N, H, D, NC, BN)(
        n0, n1, sdacc1, Wd1, r(bd1), Wd2p, bd2p, Wi1, r(bi1), Wi2, r(bi2))

    out_nodes = jnp.stack([o0[:, :OUT], o1[:, :OUT]])
    time_derivatives = jnp.stack([n0, td1])
    spatial_derivatives = jnp.stack([sd0, sd1])
    pred_inputs = jnp.stack([p0, p1])
    return (out_nodes, time_derivatives, spatial_derivatives, pred_inputs)


# speedup vs baseline: 4.7304x; 1.0512x over previous
import functools

import jax
import jax.numpy as jnp
from jax import lax
from jax.experimental import pallas as pl
from jax.experimental.pallas import tpu as pltpu
from jax.experimental.pallas import tpu_sc as plsc

F32 = jnp.float32
I32 = jnp.int32


def _edge_pass_a(E, N, H, with_hep, write_enew):
    info = plsc.get_sparse_core_info()
    NC, NS, L = info.num_cores, info.num_subcores, info.num_lanes
    NW = NC * NS
    assert E % NW == 0
    EW = E // NW
    C = 64
    nfull, tail = divmod(EW, C)
    assert nfull % 3 == 0 and nfull >= 6
    RW = -(-(N // NS) // 8) * 8
    JH = H // L
    NB = 5 if with_hep else 4

    mesh = plsc.VectorSubcoreMesh(core_axis_name="c", subcore_axis_name="s")

    outs = []
    if write_enew:
        outs.append(jax.ShapeDtypeStruct((E, H), F32))
    outs.append(jax.ShapeDtypeStruct((NC, N, H), F32))
    outs.append(jax.ShapeDtypeStruct((NC, N, H), F32))

    def bufset(cc):
        b = [
            pltpu.VMEM((2, cc), I32),
            pltpu.VMEM((cc + L,), I32),
            pltpu.VMEM((cc, H), F32),
            pltpu.VMEM((cc, H), F32),
        ]
        if with_hep:
            b.append(pltpu.VMEM((cc, H), F32))
        return b

    scratch = []
    for _ in range(3):
        scratch += bufset(C)
    if tail:
        scratch += bufset(tail)
    scratch += [
        pltpu.VMEM((16, H), F32),
        pltpu.VMEM_SHARED((N, H), F32),
        pltpu.VMEM_SHARED((N, H), F32),
    ]
    scratch += [pltpu.SemaphoreType.DMA] * 9

    @functools.partial(pl.kernel, mesh=mesh, out_type=tuple(outs),
                       scratch_types=scratch,
                       compiler_params=pltpu.CompilerParams(
                           use_tc_tiling_on_sc=False))
    def k(*refs):
        ei_h, attr_h, s_h, d_h, tab_h = refs[0:5]
        idx = 5
        if with_hep:
            hep_h = refs[idx]; idx += 1
        zeros_h = refs[idx]; idx += 1
        if write_enew:
            enew_h = refs[idx]; idx += 1
        racc_h = refs[idx]; idx += 1
        sacc_h = refs[idx]; idx += 1
        sets = [refs[idx + NB * t: idx + NB * (t + 1)] for t in range(3)]
        idx += 3 * NB
        if tail:
            tset = refs[idx:idx + NB]
            idx += NB
        tab_v, racc_sh, sacc_sh = refs[idx:idx + 3]
        idx += 3
        gsem = refs[idx:idx + 3]
        ssem = refs[idx + 3:idx + 6]
        isem = refs[idx + 6:idx + 9]

        cid = lax.axis_index("c")
        sid = lax.axis_index("s")
        wid = cid * NS + sid

        rstart = pl.multiple_of(jnp.minimum(sid * RW, N - RW), 8)
        zsl = pl.ds(rstart, RW)
        pltpu.sync_copy(zeros_h.at[zsl], racc_sh.at[zsl])
        pltpu.sync_copy(zeros_h.at[zsl], sacc_sh.at[zsl])
        pltpu.sync_copy(tab_h, tab_v)
        plsc.subcore_barrier()

        base0 = wid * EW

        def cbase(kk):
            return pl.multiple_of(
                jnp.minimum(base0 + kk * C, E - C), 8)

        def fetch_idx(kk, t):
            ci, i_t = sets[t][0], sets[t][1]
            sl = pl.ds(cbase(kk), C)
            h1 = pltpu.async_copy(ei_h.at[:, sl], ci, isem[t])
            h2 = pltpu.async_copy(attr_h.at[sl], i_t.at[pl.ds(0, C)],
                                  isem[t])
            return (h1, h2)

        def fetch(kk, t):
            ci, cr_s, cr_d = sets[t][0], sets[t][2], sets[t][3]
            sl = pl.ds(cbase(kk), C)
            out = [pltpu.async_copy(s_h.at[ci.at[0]], cr_s, gsem[t]),
                   pltpu.async_copy(d_h.at[ci.at[1]], cr_d, gsem[t])]
            if with_hep:
                out.append(pltpu.async_copy(hep_h.at[sl], sets[t][4],
                                            gsem[t]))
            return out

        def compute(t):
            i_t, cr_s, cr_d = sets[t][1], sets[t][2], sets[t][3]
            if with_hep:
                cr_h = sets[t][4]

            def row(i2, _):
                for u in range(2):
                    i = i2 * 2 + u
                    a = i_t[pl.ds(i, L)][0]
                    for j in range(JH):
                        jl = pl.ds(j * L, L)
                        v = cr_s[i, jl] + cr_d[i, jl] + tab_v[a, jl]
                        if with_hep:
                            v = v + cr_h[i, jl]
                        cr_s[i, jl] = jnp.maximum(v, 0.0)
                return 0

            lax.fori_loop(0, C // 2, row, 0)

        def scatter(kk, t):
            ci, ce_v = sets[t][0], sets[t][2]
            out = []
            if write_enew:
                out.append(pltpu.async_copy(
                    ce_v, enew_h.at[pl.ds(cbase(kk), C)], ssem[t]))
            pltpu.sync_copy(ce_v, racc_sh.at[ci.at[1]], add=True)
            pltpu.sync_copy(ce_v, sacc_sh.at[ci.at[0]], add=True)
            return out

        def group(k0):
            ihs = [fetch_idx(k0 + j, j) for j in range(3)]
            ghs = []
            for j in range(3):
                for h in ihs[j]:
                    h.wait()
                ghs.append(fetch(k0 + j, j))
            shs = []
            for j in range(3):
                for h in ghs[j]:
                    h.wait()
                compute(j)
                shs += scatter(k0 + j, j)
            for h in shs:
                h.wait()

        def body(m, _):
            group(3 * m)
            return 0

        lax.fori_loop(0, nfull // 3, body, 0)

        if tail:
            if with_hep:
                tci, ti_t, tr_s, tr_d, tr_h = tset
            else:
                tci, ti_t, tr_s, tr_d = tset
            te_v = tr_s
            sl = pl.ds(pl.multiple_of(base0 + nfull * C, 8), tail)
            pltpu.sync_copy(ei_h.at[:, sl], tci)
            pltpu.sync_copy(attr_h.at[sl], ti_t.at[pl.ds(0, tail)])
            ths = pltpu.async_copy(s_h.at[tci.at[0]], tr_s, gsem[0])
            thd = pltpu.async_copy(d_h.at[tci.at[1]], tr_d, gsem[0])
            if with_hep:
                pltpu.sync_copy(hep_h.at[sl], tr_h)
            ths.wait()
            thd.wait()

            def trow(i, _):
                a = ti_t[pl.ds(i, L)][0]
                for j in range(JH):
                    jl = pl.ds(j * L, L)
                    v = tr_s[i, jl] + tr_d[i, jl] + tab_v[a, jl]
                    if with_hep:
                        v = v + tr_h[i, jl]
                    te_v[i, jl] = jnp.maximum(v, 0.0)
                return 0

            lax.fori_loop(0, tail, trow, 0)
            pltpu.sync_copy(te_v, racc_sh.at[tci.at[1]], add=True)
            pltpu.sync_copy(te_v, sacc_sh.at[tci.at[0]], add=True)
            if write_enew:
                pltpu.sync_copy(te_v, enew_h.at[sl])

        plsc.subcore_barrier()
        osl = pl.ds(rstart, RW)
        pltpu.sync_copy(racc_sh.at[osl], racc_h.at[cid, osl])
        pltpu.sync_copy(sacc_sh.at[osl], sacc_h.at[cid, osl])

    return k


def _edge_pass_b(E, N, H):
    info = plsc.get_sparse_core_info()
    NC, NS, L = info.num_cores, info.num_subcores, info.num_lanes
    NW = NC * NS
    EW = E // NW
    C = 64
    nfull, tail = divmod(EW, C)
    assert nfull % 3 == 0 and nfull >= 6
    RW = -(-(N // NS) // 8) * 8
    JH = H // L

    mesh = plsc.VectorSubcoreMesh(core_axis_name="c", subcore_axis_name="s")

    def bufset(cc):
        return [
            pltpu.VMEM((2, cc), I32),
            pltpu.VMEM((cc + L,), F32),
            pltpu.VMEM((cc, H), F32),
        ]

    scratch = []
    for _ in range(3):
        scratch += bufset(C)
    if tail:
        scratch += bufset(tail)
    scratch += [
        pltpu.VMEM((L,), F32),
        pltpu.VMEM_SHARED((N, H), F32),
    ]
    scratch += [pltpu.SemaphoreType.DMA] * 9

    @functools.partial(
        pl.kernel, mesh=mesh,
        out_type=jax.ShapeDtypeStruct((NC, N, H), F32),
        scratch_types=scratch,
        compiler_params=pltpu.CompilerParams(use_tc_tiling_on_sc=False))
    def k(*refs):
        ei_h, spl_h, coeff_h, nn_h, zeros_h, sd_h = refs[0:6]
        idx = 6
        sets = [refs[idx + 3 * t: idx + 3 * (t + 1)] for t in range(3)]
        idx += 9
        if tail:
            tset = refs[idx:idx + 3]
            idx += 3
        c_v, sd_sh = refs[idx:idx + 2]
        idx += 2
        gsem = refs[idx:idx + 3]
        ssem = refs[idx + 3:idx + 6]
        isem = refs[idx + 6:idx + 9]

        cid = lax.axis_index("c")
        sid = lax.axis_index("s")
        wid = cid * NS + sid

        rstart = pl.multiple_of(jnp.minimum(sid * RW, N - RW), 8)
        zsl = pl.ds(rstart, RW)
        pltpu.sync_copy(zeros_h.at[zsl], sd_sh.at[zsl])
        pltpu.sync_copy(coeff_h, c_v)
        plsc.subcore_barrier()

        coef = c_v[...][0]
        base0 = wid * EW

        def cbase(kk):
            return pl.multiple_of(
                jnp.minimum(base0 + kk * C, E - C), 8)

        def fetch_idx(kk, t):
            ci, cv_l, cr_n = sets[t]
            sl = pl.ds(cbase(kk), C)
            return pltpu.async_copy(ei_h.at[:, sl], ci, isem[t])

        def fetch(kk, t):
            ci, cv_l, cr_n = sets[t]
            sl = pl.ds(cbase(kk), C)
            hl = pltpu.async_copy(spl_h.at[sl], cv_l.at[pl.ds(0, C)],
                                  gsem[t])
            hn = pltpu.async_copy(nn_h.at[ci.at[0]], cr_n, gsem[t])
            return (hl, hn)

        def compute(t):
            ci, cv_l, cr_n = sets[t]

            def row(i2, _):
                for u in range(2):
                    i = i2 * 2 + u
                    s = cv_l[pl.ds(i, L)][0] * coef
                    for j in range(JH):
                        jl = pl.ds(j * L, L)
                        cr_n[i, jl] = cr_n[i, jl] * s
                return 0

            lax.fori_loop(0, C // 2, row, 0)

        def scatter(t):
            ci, cv_l, cr_n = sets[t]
            pltpu.sync_copy(cr_n, sd_sh.at[ci.at[1]], add=True)

        def group(k0):
            ihs = [fetch_idx(k0 + j, j) for j in range(3)]
            ghs = []
            for j in range(3):
                ihs[j].wait()
                ghs.append(fetch(k0 + j, j))
            for j in range(3):
                for h in ghs[j]:
                    h.wait()
                compute(j)
                scatter(j)

        def body(m, _):
            group(3 * m)
            return 0

        lax.fori_loop(0, nfull // 3, body, 0)

        if tail:
            tci, tv_l, tr_n = tset
            sl = pl.ds(pl.multiple_of(base0 + nfull * C, 8), tail)
            pltpu.sync_copy(ei_h.at[:, sl], tci)
            pltpu.sync_copy(spl_h.at[sl], tv_l.at[pl.ds(0, tail)])
            pltpu.async_copy(nn_h.at[tci.at[0]], tr_n, gsem[0]).wait()

            def trow(i, _):
                s = tv_l[pl.ds(i, L)][0] * coef
                for j in range(JH):
                    jl = pl.ds(j * L, L)
                    tr_n[i, jl] = tr_n[i, jl] * s
                return 0

            lax.fori_loop(0, tail, trow, 0)
            pltpu.sync_copy(tr_n, sd_sh.at[tci.at[1]], add=True)

        plsc.subcore_barrier()
        osl = pl.ds(rstart, RW)
        pltpu.sync_copy(sd_sh.at[osl], sd_h.at[cid, osl])

    return k


def _dot(a, b):
    return jnp.dot(a, b, preferred_element_type=F32)


def _k1_prep(N, D, H, K, BN):
    ng = N // BN

    def body(x0, x1, wenc, benc, wixs, wixd, emb, wie, wg, beb, g0, wng, bnb,
             ix0, ix1, s0, d0, embw, table0, nbias0):
        a0 = jnp.maximum(_dot(x0[...], wenc[...]) + benc[...], 0.0)
        a1 = jnp.maximum(_dot(x1[...], wenc[...]) + benc[...], 0.0)
        ix0[...] = a0
        ix1[...] = a1
        s0[...] = _dot(a0, wixs[...])
        d0[...] = _dot(a0, wixd[...])
        ew = _dot(emb[...], wie[...])
        embw[...] = ew
        gv = _dot(g0[...], wg[...]) + beb[...]
        table0[...] = ew + gv
        nbias0[...] = bnb[...] + _dot(g0[...], wng[...])

    full = lambda i: (0, 0)
    blk = lambda i: (i, 0)
    return pl.pallas_call(
        body,
        grid=(ng,),
        in_specs=[
            pl.BlockSpec((BN, D), blk), pl.BlockSpec((BN, D), blk),
            pl.BlockSpec((D, H), full), pl.BlockSpec((1, H), full),
            pl.BlockSpec((H, H), full), pl.BlockSpec((H, H), full),
            pl.BlockSpec((K, H), full), pl.BlockSpec((H, H), full),
            pl.BlockSpec((H, H), full), pl.BlockSpec((1, H), full),
            pl.BlockSpec((1, H), full), pl.BlockSpec((H, H), full),
            pl.BlockSpec((1, H), full),
        ],
        out_specs=[
            pl.BlockSpec((BN, H), blk), pl.BlockSpec((BN, H), blk),
            pl.BlockSpec((BN, H), blk), pl.BlockSpec((BN, H), blk),
            pl.BlockSpec((K, H), full), pl.BlockSpec((K, H), full),
            pl.BlockSpec((1, H), full),
        ],
        out_shape=[
            jax.ShapeDtypeStruct((N, H), F32), jax.ShapeDtypeStruct((N, H), F32),
            jax.ShapeDtypeStruct((N, H), F32), jax.ShapeDtypeStruct((N, H), F32),
            jax.ShapeDtypeStruct((K, H), F32), jax.ShapeDtypeStruct((K, H), F32),
            jax.ShapeDtypeStruct((1, H), F32),
        ],
    )


def _k2_node0(N, E, H, K, NC, BN):
    ng = N // BN

    def body(ix0, ix1, racc, sacc, wnix, wnrecv, wnsent, nbias0,
             whxs, whxd, wixs, wixd, embw, wg, beb, wgb, bgb, wng, bnb, g0,
             n0, sn1, d1, table1, nbias1, nsum, rsum):
        i = pl.program_id(0)
        recv = racc[0] + racc[1]
        sent = sacc[0] + sacc[1]
        a0 = jnp.maximum(
            _dot(ix0[...], wnix[...]) + _dot(recv, wnrecv[...])
            + _dot(sent, wnsent[...]) + nbias0[...], 0.0)
        n0[...] = a0
        sn1[...] = _dot(a0, whxs[...]) + _dot(ix1[...], wixs[...])
        d1[...] = _dot(a0, whxd[...]) + _dot(ix1[...], wixd[...])

        @pl.when(i == 0)
        def _():
            nsum[...] = jnp.zeros_like(nsum)
            rsum[...] = jnp.zeros_like(rsum)

        nsum[...] += jnp.sum(a0, axis=0, keepdims=True)
        rsum[...] += jnp.sum(recv, axis=0, keepdims=True)

        @pl.when(i == ng - 1)
        def _():
            n_mean = nsum[...] / float(N)
            e_mean = rsum[...] / float(E)
            g_in = jnp.concatenate([n_mean, e_mean, g0[...]], axis=1)
            g1 = jnp.maximum(_dot(g_in, wgb[...]) + bgb[...], 0.0)
            table1[...] = embw[...] + _dot(g1, wg[...]) + beb[...]
            nbias1[...] = bnb[...] + _dot(g1, wng[...])

    full = lambda i: (0, 0)
    blk = lambda i: (i, 0)
    blk3 = lambda i: (0, i, 0)
    return pl.pallas_call(
        body,
        grid=(ng,),
        in_specs=[
            pl.BlockSpec((BN, H), blk), pl.BlockSpec((BN, H), blk),
            pl.BlockSpec((NC, BN, H), blk3), pl.BlockSpec((NC, BN, H), blk3),
            pl.BlockSpec((H, H), full), pl.BlockSpec((H, H), full),
            pl.BlockSpec((H, H), full), pl.BlockSpec((1, H), full),
            pl.BlockSpec((H, H), full), pl.BlockSpec((H, H), full),
            pl.BlockSpec((H, H), full), pl.BlockSpec((H, H), full),
            pl.BlockSpec((K, H), full), pl.BlockSpec((H, H), full),
            pl.BlockSpec((1, H), full), pl.BlockSpec((3 * H, H), full),
            pl.BlockSpec((1, H), full), pl.BlockSpec((H, H), full),
            pl.BlockSpec((1, H), full), pl.BlockSpec((1, H), full),
        ],
        out_specs=[
            pl.BlockSpec((BN, H), blk), pl.BlockSpec((BN, H), blk),
            pl.BlockSpec((BN, H), blk), pl.BlockSpec((K, H), full),
            pl.BlockSpec((1, H), full), pl.BlockSpec((1, H), full),
            pl.BlockSpec((1, H), full),
        ],
        out_shape=[
            jax.ShapeDtypeStruct((N, H), F32),
            jax.ShapeDtypeStruct((N, H), F32),
            jax.ShapeDtypeStruct((N, H), F32), jax.ShapeDtypeStruct((K, H), F32),
            jax.ShapeDtypeStruct((1, H), F32), jax.ShapeDtypeStruct((1, H), F32),
            jax.ShapeDtypeStruct((1, H), F32),
        ],
    )


def _k3_heproj(E, H, BE):
    ng = E // BE

    def body(e0, whe, out):
        out[...] = _dot(e0[...], whe[...])

    return pl.pallas_call(
        body,
        grid=(ng,),
        in_specs=[pl.BlockSpec((BE, H), lambda i: (i, 0)),
                  pl.BlockSpec((H, H), lambda i: (0, 0))],
        out_specs=pl.BlockSpec((BE, H), lambda i: (i, 0)),
        out_shape=jax.ShapeDtypeStruct((E, H), F32),
    )


def _k4_node1(N, H, NC, BN):
    ng = N // BN

    def body(n0, ix1, racc, sacc, wnhx, wnix, wnrecv, wnsent, nbias1, sdacc0,
             n1, td1, sd0):
        recv = racc[0] + racc[1]
        sent = sacc[0] + sacc[1]
        a1 = jnp.maximum(
            _dot(n0[...], wnhx[...]) + _dot(ix1[...], wnix[...])
            + _dot(recv, wnrecv[...]) + _dot(sent, wnsent[...])
            + nbias1[...], 0.0)
        n1[...] = a1
        td1[...] = a1 - n0[...]
        sd0[...] = sdacc0[0] + sdacc0[1]

    full = lambda i: (0, 0)
    blk = lambda i: (i, 0)
    blk3 = lambda i: (0, i, 0)
    return pl.pallas_call(
        body,
        grid=(ng,),
        in_specs=[
            pl.BlockSpec((BN, H), blk), pl.BlockSpec((BN, H), blk),
            pl.BlockSpec((NC, BN, H), blk3), pl.BlockSpec((NC, BN, H), blk3),
            pl.BlockSpec((H, H), full), pl.BlockSpec((H, H), full),
            pl.BlockSpec((H, H), full), pl.BlockSpec((H, H), full),
            pl.BlockSpec((1, H), full), pl.BlockSpec((NC, BN, H), blk3),
        ],
        out_specs=[pl.BlockSpec((BN, H), blk), pl.BlockSpec((BN, H), blk),
                   pl.BlockSpec((BN, H), blk)],
        out_shape=[jax.ShapeDtypeStruct((N, H), F32),
                   jax.ShapeDtypeStruct((N, H), F32),
                   jax.ShapeDtypeStruct((N, H), F32)],
    )


def _k5_dec(N, H, D, NC, BN):
    ng = N // BN

    def body(n0, n1, sdacc1, wd1, bd1, wd2p, bd2p, wi1, bi1, wi2, bi2,
             o0, o1, p0, p1, sd1):
        h00 = jnp.maximum(_dot(n0[...], wd1[...]) + bd1[...], 0.0)
        h01 = jnp.maximum(_dot(n1[...], wd1[...]) + bd1[...], 0.0)
        o0[...] = _dot(h00, wd2p[...]) + bd2p[...]
        o1[...] = _dot(h01, wd2p[...]) + bd2p[...]
        h10 = jnp.maximum(_dot(n0[...], wi1[...]) + bi1[...], 0.0)
        h11 = jnp.maximum(_dot(n1[...], wi1[...]) + bi1[...], 0.0)
        p0[...] = _dot(h10, wi2[...]) + bi2[...]
        p1[...] = _dot(h11, wi2[...]) + bi2[...]
        sd1[...] = sdacc1[0] + sdacc1[1]

    full = lambda i: (0, 0)
    blk = lambda i: (i, 0)
    blk3 = lambda i: (0, i, 0)
    return pl.pallas_call(
        body,
        grid=(ng,),
        in_specs=[
            pl.BlockSpec((BN, H), blk), pl.BlockSpec((BN, H), blk),
            pl.BlockSpec((NC, BN, H), blk3),
            pl.BlockSpec((H, H), full), pl.BlockSpec((1, H), full),
            pl.BlockSpec((H, D), full), pl.BlockSpec((1, D), full),
            pl.BlockSpec((H, H), full), pl.BlockSpec((1, H), full),
            pl.BlockSpec((H, D), full), pl.BlockSpec((1, D), full),
        ],
        out_specs=[pl.BlockSpec((BN, D), blk), pl.BlockSpec((BN, D), blk),
                   pl.BlockSpec((BN, D), blk), pl.BlockSpec((BN, D), blk),
                   pl.BlockSpec((BN, H), blk)],
        out_shape=[jax.ShapeDtypeStruct((N, D), F32),
                   jax.ShapeDtypeStruct((N, D), F32),
                   jax.ShapeDtypeStruct((N, D), F32),
                   jax.ShapeDtypeStruct((N, D), F32),
                   jax.ShapeDtypeStruct((N, H), F32)],
    )


def kernel(x, edge_index, edge_attr, global_attr, sp_L_values, coeff,
           num_processing_steps, emb, Wenc, benc, Web, beb, Wnb, bnb, Wgb,
           bgb, Wd1, bd1, Wd2, bd2, Wi1, bi1, Wi2, bi2):
    T, N, D = x.shape
    E = edge_index.shape[1]
    H = Wenc.shape[1]
    K = emb.shape[0]
    OUT = Wd2.shape[1]
    assert T == 2

    info = plsc.get_sparse_core_info()
    NC = info.num_cores
    BN = 1000

    BE = 2000

    W_he, W_hxs, W_hxd, W_ie, W_ixs, W_ixd, W_g = (
        Web[i * H:(i + 1) * H] for i in range(7))
    Wn_hx, Wn_ix, Wn_recv, Wn_sent, Wn_g = (
        Wnb[i * H:(i + 1) * H] for i in range(5))

    g0 = global_attr
    r = lambda v: v.reshape(1, -1)
    zeros_nh = jnp.zeros((N, H), F32)
    coeff16 = jnp.concatenate([coeff, jnp.zeros((15,), F32)])
    Wd2p = jnp.pad(Wd2, ((0, 0), (0, D - OUT)))
    bd2p = jnp.pad(bd2, (0, D - OUT)).reshape(1, D)

    ix0, ix1, S0, D0, embW, table0, nbias0 = _k1_prep(N, D, H, K, BN)(
        x[0], x[1], Wenc, r(benc), W_ixs, W_ixd, emb, W_ie, W_g, r(beb),
        g0, Wn_g, r(bnb))

    e0, racc0, sacc0 = _edge_pass_a(E, N, H, with_hep=False,
                                    write_enew=True)(
        edge_index, edge_attr[0], S0, D0, table0, zeros_nh)

    n0, S1, D1, table1, nbias1, _, _ = _k2_node0(N, E, H, K, NC, BN)(
        ix0, ix1, racc0, sacc0, Wn_ix, Wn_recv, Wn_sent, nbias0,
        W_hxs, W_hxd, W_ixs, W_ixd, embW, W_g, r(beb), Wgb, r(bgb),
        Wn_g, r(bnb), g0)

    sdacc0 = _edge_pass_b(E, N, H)(edge_index, sp_L_values, coeff16, n0,
                                   zeros_nh)

    hep1 = _k3_heproj(E, H, BE)(e0, W_he)

    racc1, sacc1 = _edge_pass_a(E, N, H, with_hep=True,
                                write_enew=False)(
        edge_index, edge_attr[1], S1, D1, table1, hep1, zeros_nh)

    n1, td1, sd0 = _k4_node1(N, H, NC, BN)(
        n0, ix1, racc1, sacc1, Wn_hx, Wn_ix, Wn_recv, Wn_sent, nbias1,
        sdacc0)

    sdacc1 = _edge_pass_b(E, N, H)(edge_index, sp_L_values, coeff16, n1,
                                   zeros_nh)

    o0, o1, p0, p1, sd1 = _k5_dec(N, H, D, NC, BN)(
        n0, n1, sdacc1, Wd1, r(bd1), Wd2p, bd2p, Wi1, r(bi1), Wi2, r(bi2))

    out_nodes = jnp.stack([o0[:, :OUT], o1[:, :OUT]])
    time_derivatives = jnp.stack([n0, td1])
    spatial_derivatives = jnp.stack([sd0, sd1])
    pred_inputs = jnp.stack([p0, p1])
    return (out_nodes, time_derivatives, spatial_derivatives, pred_inputs)


# speedup vs baseline: 4.8617x; 1.0278x over previous
import functools

import jax
import jax.numpy as jnp
from jax import lax
from jax.experimental import pallas as pl
from jax.experimental.pallas import tpu as pltpu
from jax.experimental.pallas import tpu_sc as plsc

F32 = jnp.float32
I32 = jnp.int32


def _edge_pass_a(E, N, H, with_hep, write_enew):
    info = plsc.get_sparse_core_info()
    NC, NS, L = info.num_cores, info.num_subcores, info.num_lanes
    NW = NC * NS
    assert E % NW == 0
    EW = E // NW
    C = 64
    nfull, tail = divmod(EW, C)
    assert nfull % 3 == 0 and nfull >= 6
    RW = -(-(N // NS) // 8) * 8
    JH = H // L
    NB = 5 if with_hep else 4

    mesh = plsc.VectorSubcoreMesh(core_axis_name="c", subcore_axis_name="s")

    outs = []
    if write_enew:
        outs.append(jax.ShapeDtypeStruct((E, H), F32))
    outs.append(jax.ShapeDtypeStruct((NC, N, H), F32))
    outs.append(jax.ShapeDtypeStruct((NC, N, H), F32))

    def bufset(cc):
        b = [
            pltpu.VMEM((2, cc), I32),
            pltpu.VMEM((cc + L,), I32),
            pltpu.VMEM((cc, H), F32),
            pltpu.VMEM((cc, H), F32),
        ]
        if with_hep:
            b.append(pltpu.VMEM((cc, H), F32))
        return b

    scratch = []
    for _ in range(3):
        scratch += bufset(C)
    if tail:
        scratch += bufset(tail)
    scratch += [
        pltpu.VMEM((16, H), F32),
        pltpu.VMEM_SHARED((N, H), F32),
        pltpu.VMEM_SHARED((N, H), F32),
    ]
    scratch += [pltpu.SemaphoreType.DMA] * 9

    @functools.partial(pl.kernel, mesh=mesh, out_type=tuple(outs),
                       scratch_types=scratch,
                       compiler_params=pltpu.CompilerParams(
                           use_tc_tiling_on_sc=False))
    def k(*refs):
        ei_h, attr_h, s_h, d_h, tab_h = refs[0:5]
        idx = 5
        if with_hep:
            hep_h = refs[idx]; idx += 1
        zeros_h = refs[idx]; idx += 1
        if write_enew:
            enew_h = refs[idx]; idx += 1
        racc_h = refs[idx]; idx += 1
        sacc_h = refs[idx]; idx += 1
        sets = [refs[idx + NB * t: idx + NB * (t + 1)] for t in range(3)]
        idx += 3 * NB
        if tail:
            tset = refs[idx:idx + NB]
            idx += NB
        tab_v, racc_sh, sacc_sh = refs[idx:idx + 3]
        idx += 3
        gsem = refs[idx:idx + 3]
        ssem = refs[idx + 3:idx + 6]
        isem = refs[idx + 6:idx + 9]

        cid = lax.axis_index("c")
        sid = lax.axis_index("s")
        wid = cid * NS + sid

        rstart = pl.multiple_of(jnp.minimum(sid * RW, N - RW), 8)
        zsl = pl.ds(rstart, RW)
        pltpu.sync_copy(zeros_h.at[zsl], racc_sh.at[zsl])
        pltpu.sync_copy(zeros_h.at[zsl], sacc_sh.at[zsl])
        pltpu.sync_copy(tab_h, tab_v)
        plsc.subcore_barrier()

        base0 = wid * EW

        def cbase(kk):
            return pl.multiple_of(
                jnp.minimum(base0 + kk * C, E - C), 8)

        def fetch_idx(kk, t):
            ci, i_t = sets[t][0], sets[t][1]
            sl = pl.ds(cbase(kk), C)
            h1 = pltpu.async_copy(ei_h.at[:, sl], ci, isem[t])
            h2 = pltpu.async_copy(attr_h.at[sl], i_t.at[pl.ds(0, C)],
                                  isem[t])
            return (h1, h2)

        def fetch(kk, t):
            ci, cr_s, cr_d = sets[t][0], sets[t][2], sets[t][3]
            sl = pl.ds(cbase(kk), C)
            out = [pltpu.async_copy(s_h.at[ci.at[0]], cr_s, gsem[t]),
                   pltpu.async_copy(d_h.at[ci.at[1]], cr_d, gsem[t])]
            if with_hep:
                out.append(pltpu.async_copy(hep_h.at[sl], sets[t][4],
                                            gsem[t]))
            return out

        def compute(t):
            i_t, cr_s, cr_d = sets[t][1], sets[t][2], sets[t][3]
            if with_hep:
                cr_h = sets[t][4]

            def row(i2, _):
                for u in range(2):
                    i = i2 * 2 + u
                    a = i_t[pl.ds(i, L)][0]
                    for j in range(JH):
                        jl = pl.ds(j * L, L)
                        v = cr_s[i, jl] + cr_d[i, jl] + tab_v[a, jl]
                        if with_hep:
                            v = v + cr_h[i, jl]
                        cr_s[i, jl] = jnp.maximum(v, 0.0)
                return 0

            lax.fori_loop(0, C // 2, row, 0)

        def scatter(kk, t):
            ci, ce_v = sets[t][0], sets[t][2]
            out = []
            if write_enew:
                out.append(pltpu.async_copy(
                    ce_v, enew_h.at[pl.ds(cbase(kk), C)], ssem[t]))
            pltpu.sync_copy(ce_v, racc_sh.at[ci.at[1]], add=True)
            pltpu.sync_copy(ce_v, sacc_sh.at[ci.at[0]], add=True)
            return out

        def group(k0):
            ihs = [fetch_idx(k0 + j, j) for j in range(3)]
            ghs = []
            for j in range(3):
                for h in ihs[j]:
                    h.wait()
                ghs.append(fetch(k0 + j, j))
            shs = []
            for j in range(3):
                for h in ghs[j]:
                    h.wait()
                compute(j)
                shs += scatter(k0 + j, j)
            for h in shs:
                h.wait()

        def body(m, _):
            group(3 * m)
            return 0

        lax.fori_loop(0, nfull // 3, body, 0)

        if tail:
            if with_hep:
                tci, ti_t, tr_s, tr_d, tr_h = tset
            else:
                tci, ti_t, tr_s, tr_d = tset
            te_v = tr_s
            sl = pl.ds(pl.multiple_of(base0 + nfull * C, 8), tail)
            pltpu.sync_copy(ei_h.at[:, sl], tci)
            pltpu.sync_copy(attr_h.at[sl], ti_t.at[pl.ds(0, tail)])
            ths = pltpu.async_copy(s_h.at[tci.at[0]], tr_s, gsem[0])
            thd = pltpu.async_copy(d_h.at[tci.at[1]], tr_d, gsem[0])
            if with_hep:
                pltpu.sync_copy(hep_h.at[sl], tr_h)
            ths.wait()
            thd.wait()

            def trow(i, _):
                a = ti_t[pl.ds(i, L)][0]
                for j in range(JH):
                    jl = pl.ds(j * L, L)
                    v = tr_s[i, jl] + tr_d[i, jl] + tab_v[a, jl]
                    if with_hep:
                        v = v + tr_h[i, jl]
                    te_v[i, jl] = jnp.maximum(v, 0.0)
                return 0

            lax.fori_loop(0, tail, trow, 0)
            pltpu.sync_copy(te_v, racc_sh.at[tci.at[1]], add=True)
            pltpu.sync_copy(te_v, sacc_sh.at[tci.at[0]], add=True)
            if write_enew:
                pltpu.sync_copy(te_v, enew_h.at[sl])

        plsc.subcore_barrier()
        osl = pl.ds(rstart, RW)
        pltpu.sync_copy(racc_sh.at[osl], racc_h.at[cid, osl])
        pltpu.sync_copy(sacc_sh.at[osl], sacc_h.at[cid, osl])

    return k


def _edge_pass_b(E, N, H):
    info = plsc.get_sparse_core_info()
    NC, NS, L = info.num_cores, info.num_subcores, info.num_lanes
    NW = NC * NS
    EW = E // NW
    C = 64
    nfull, tail = divmod(EW, C)
    assert nfull % 3 == 0 and nfull >= 6
    RW = -(-(N // NS) // 8) * 8
    JH = H // L

    mesh = plsc.VectorSubcoreMesh(core_axis_name="c", subcore_axis_name="s")

    def bufset(cc):
        return [
            pltpu.VMEM((2, cc), I32),
            pltpu.VMEM((cc + L,), F32),
            pltpu.VMEM((cc, H), F32),
        ]

    scratch = []
    for _ in range(3):
        scratch += bufset(C)
    if tail:
        scratch += bufset(tail)
    scratch += [
        pltpu.VMEM((L,), F32),
        pltpu.VMEM_SHARED((N, H), F32),
    ]
    scratch += [pltpu.SemaphoreType.DMA] * 9

    @functools.partial(
        pl.kernel, mesh=mesh,
        out_type=jax.ShapeDtypeStruct((NC, N, H), F32),
        scratch_types=scratch,
        compiler_params=pltpu.CompilerParams(use_tc_tiling_on_sc=False))
    def k(*refs):
        ei_h, spl_h, coeff_h, nn_h, zeros_h, sd_h = refs[0:6]
        idx = 6
        sets = [refs[idx + 3 * t: idx + 3 * (t + 1)] for t in range(3)]
        idx += 9
        if tail:
            tset = refs[idx:idx + 3]
            idx += 3
        c_v, sd_sh = refs[idx:idx + 2]
        idx += 2
        gsem = refs[idx:idx + 3]
        ssem = refs[idx + 3:idx + 6]
        isem = refs[idx + 6:idx + 9]

        cid = lax.axis_index("c")
        sid = lax.axis_index("s")
        wid = cid * NS + sid

        rstart = pl.multiple_of(jnp.minimum(sid * RW, N - RW), 8)
        zsl = pl.ds(rstart, RW)
        pltpu.sync_copy(zeros_h.at[zsl], sd_sh.at[zsl])
        pltpu.sync_copy(coeff_h, c_v)
        plsc.subcore_barrier()

        coef = c_v[...][0]
        base0 = wid * EW

        def cbase(kk):
            return pl.multiple_of(
                jnp.minimum(base0 + kk * C, E - C), 8)

        def fetch_idx(kk, t):
            ci, cv_l, cr_n = sets[t]
            sl = pl.ds(cbase(kk), C)
            return pltpu.async_copy(ei_h.at[:, sl], ci, isem[t])

        def fetch(kk, t):
            ci, cv_l, cr_n = sets[t]
            sl = pl.ds(cbase(kk), C)
            hl = pltpu.async_copy(spl_h.at[sl], cv_l.at[pl.ds(0, C)],
                                  gsem[t])
            hn = pltpu.async_copy(nn_h.at[ci.at[0]], cr_n, gsem[t])
            return (hl, hn)

        def compute(t):
            ci, cv_l, cr_n = sets[t]

            def row(i2, _):
                for u in range(2):
                    i = i2 * 2 + u
                    s = cv_l[pl.ds(i, L)][0] * coef
                    for j in range(JH):
                        jl = pl.ds(j * L, L)
                        cr_n[i, jl] = cr_n[i, jl] * s
                return 0

            lax.fori_loop(0, C // 2, row, 0)

        def scatter(t):
            ci, cv_l, cr_n = sets[t]
            pltpu.sync_copy(cr_n, sd_sh.at[ci.at[1]], add=True)

        def group(k0):
            ihs = [fetch_idx(k0 + j, j) for j in range(3)]
            ghs = []
            for j in range(3):
                ihs[j].wait()
                ghs.append(fetch(k0 + j, j))
            for j in range(3):
                for h in ghs[j]:
                    h.wait()
                compute(j)
                scatter(j)

        def body(m, _):
            group(3 * m)
            return 0

        lax.fori_loop(0, nfull // 3, body, 0)

        if tail:
            tci, tv_l, tr_n = tset
            sl = pl.ds(pl.multiple_of(base0 + nfull * C, 8), tail)
            pltpu.sync_copy(ei_h.at[:, sl], tci)
            pltpu.sync_copy(spl_h.at[sl], tv_l.at[pl.ds(0, tail)])
            pltpu.async_copy(nn_h.at[tci.at[0]], tr_n, gsem[0]).wait()

            def trow(i, _):
                s = tv_l[pl.ds(i, L)][0] * coef
                for j in range(JH):
                    jl = pl.ds(j * L, L)
                    tr_n[i, jl] = tr_n[i, jl] * s
                return 0

            lax.fori_loop(0, tail, trow, 0)
            pltpu.sync_copy(tr_n, sd_sh.at[tci.at[1]], add=True)

        plsc.subcore_barrier()
        osl = pl.ds(rstart, RW)
        pltpu.sync_copy(sd_sh.at[osl], sd_h.at[cid, osl])

    return k


def _dot(a, b):
    return jnp.dot(a, b, preferred_element_type=F32)


def _k1_prep(N, D, H, K, BN):
    ng = N // BN

    def body(x0, x1, wenc, benc, wixs, wixd, emb, wie, wg, beb, g0, wng, bnb,
             ix0, ix1, s0, d0, embw, table0, nbias0):
        a0 = jnp.maximum(_dot(x0[...], wenc[...]) + benc[...], 0.0)
        a1 = jnp.maximum(_dot(x1[...], wenc[...]) + benc[...], 0.0)
        ix0[...] = a0
        ix1[...] = a1
        s0[...] = _dot(a0, wixs[...])
        d0[...] = _dot(a0, wixd[...])
        ew = _dot(emb[...], wie[...])
        embw[...] = ew
        gv = _dot(g0[...], wg[...]) + beb[...]
        table0[...] = ew + gv
        nbias0[...] = bnb[...] + _dot(g0[...], wng[...])

    full = lambda i: (0, 0)
    blk = lambda i: (i, 0)
    return pl.pallas_call(
        body,
        grid=(ng,),
        in_specs=[
            pl.BlockSpec((BN, D), blk), pl.BlockSpec((BN, D), blk),
            pl.BlockSpec((D, H), full), pl.BlockSpec((1, H), full),
            pl.BlockSpec((H, H), full), pl.BlockSpec((H, H), full),
            pl.BlockSpec((K, H), full), pl.BlockSpec((H, H), full),
            pl.BlockSpec((H, H), full), pl.BlockSpec((1, H), full),
            pl.BlockSpec((1, H), full), pl.BlockSpec((H, H), full),
            pl.BlockSpec((1, H), full),
        ],
        out_specs=[
            pl.BlockSpec((BN, H), blk), pl.BlockSpec((BN, H), blk),
            pl.BlockSpec((BN, H), blk), pl.BlockSpec((BN, H), blk),
            pl.BlockSpec((K, H), full), pl.BlockSpec((K, H), full),
            pl.BlockSpec((1, H), full),
        ],
        out_shape=[
            jax.ShapeDtypeStruct((N, H), F32), jax.ShapeDtypeStruct((N, H), F32),
            jax.ShapeDtypeStruct((N, H), F32), jax.ShapeDtypeStruct((N, H), F32),
            jax.ShapeDtypeStruct((K, H), F32), jax.ShapeDtypeStruct((K, H), F32),
            jax.ShapeDtypeStruct((1, H), F32),
        ],
    )


def _k2_node0(N, E, H, K, NC, BN):
    ng = N // BN

    def body(ix0, ix1, racc, sacc, wnix, wnrecv, wnsent, nbias0,
             whxs, whxd, wixs, wixd, embw, wg, beb, wgb, bgb, wng, bnb, g0,
             n0, sn1, d1, table1, nbias1, nsum, rsum):
        i = pl.program_id(0)
        recv = racc[0] + racc[1]
        sent = sacc[0] + sacc[1]
        a0 = jnp.maximum(
            _dot(ix0[...], wnix[...]) + _dot(recv, wnrecv[...])
            + _dot(sent, wnsent[...]) + nbias0[...], 0.0)
        n0[...] = a0
        sn1[...] = _dot(a0, whxs[...]) + _dot(ix1[...], wixs[...])
        d1[...] = _dot(a0, whxd[...]) + _dot(ix1[...], wixd[...])

        @pl.when(i == 0)
        def _():
            nsum[...] = jnp.zeros_like(nsum)
            rsum[...] = jnp.zeros_like(rsum)

        nsum[...] += jnp.sum(a0, axis=0, keepdims=True)
        rsum[...] += jnp.sum(recv, axis=0, keepdims=True)

        @pl.when(i == ng - 1)
        def _():
            n_mean = nsum[...] / float(N)
            e_mean = rsum[...] / float(E)
            g_in = jnp.concatenate([n_mean, e_mean, g0[...]], axis=1)
            g1 = jnp.maximum(_dot(g_in, wgb[...]) + bgb[...], 0.0)
            table1[...] = embw[...] + _dot(g1, wg[...]) + beb[...]
            nbias1[...] = bnb[...] + _dot(g1, wng[...])

    full = lambda i: (0, 0)
    blk = lambda i: (i, 0)
    blk3 = lambda i: (0, i, 0)
    return pl.pallas_call(
        body,
        grid=(ng,),
        in_specs=[
            pl.BlockSpec((BN, H), blk), pl.BlockSpec((BN, H), blk),
            pl.BlockSpec((NC, BN, H), blk3), pl.BlockSpec((NC, BN, H), blk3),
            pl.BlockSpec((H, H), full), pl.BlockSpec((H, H), full),
            pl.BlockSpec((H, H), full), pl.BlockSpec((1, H), full),
            pl.BlockSpec((H, H), full), pl.BlockSpec((H, H), full),
            pl.BlockSpec((H, H), full), pl.BlockSpec((H, H), full),
            pl.BlockSpec((K, H), full), pl.BlockSpec((H, H), full),
            pl.BlockSpec((1, H), full), pl.BlockSpec((3 * H, H), full),
            pl.BlockSpec((1, H), full), pl.BlockSpec((H, H), full),
            pl.BlockSpec((1, H), full), pl.BlockSpec((1, H), full),
        ],
        out_specs=[
            pl.BlockSpec((BN, H), blk), pl.BlockSpec((BN, H), blk),
            pl.BlockSpec((BN, H), blk), pl.BlockSpec((K, H), full),
            pl.BlockSpec((1, H), full), pl.BlockSpec((1, H), full),
            pl.BlockSpec((1, H), full),
        ],
        out_shape=[
            jax.ShapeDtypeStruct((N, H), F32),
            jax.ShapeDtypeStruct((N, H), F32),
            jax.ShapeDtypeStruct((N, H), F32), jax.ShapeDtypeStruct((K, H), F32),
            jax.ShapeDtypeStruct((1, H), F32), jax.ShapeDtypeStruct((1, H), F32),
            jax.ShapeDtypeStruct((1, H), F32),
        ],
    )


def _k3_heproj(E, H, BE):
    ng = E // BE

    def body(e0, whe, out):
        out[...] = _dot(e0[...], whe[...])

    return pl.pallas_call(
        body,
        grid=(ng,),
        in_specs=[pl.BlockSpec((BE, H), lambda i: (i, 0)),
                  pl.BlockSpec((H, H), lambda i: (0, 0))],
        out_specs=pl.BlockSpec((BE, H), lambda i: (i, 0)),
        out_shape=jax.ShapeDtypeStruct((E, H), F32),
    )


def _k4_node1(N, H, NC, BN):
    ng = N // BN

    def body(n0, ix1, racc, sacc, wnhx, wnix, wnrecv, wnsent, nbias1, sdacc0,
             n1, td1, sd0):
        recv = racc[0] + racc[1]
        sent = sacc[0] + sacc[1]
        a1 = jnp.maximum(
            _dot(n0[...], wnhx[...]) + _dot(ix1[...], wnix[...])
            + _dot(recv, wnrecv[...]) + _dot(sent, wnsent[...])
            + nbias1[...], 0.0)
        n1[...] = a1
        td1[...] = a1 - n0[...]
        sd0[...] = sdacc0[0] + sdacc0[1]

    full = lambda i: (0, 0)
    blk = lambda i: (i, 0)
    blk3 = lambda i: (0, i, 0)
    return pl.pallas_call(
        body,
        grid=(ng,),
        in_specs=[
            pl.BlockSpec((BN, H), blk), pl.BlockSpec((BN, H), blk),
            pl.BlockSpec((NC, BN, H), blk3), pl.BlockSpec((NC, BN, H), blk3),
            pl.BlockSpec((H, H), full), pl.BlockSpec((H, H), full),
            pl.BlockSpec((H, H), full), pl.BlockSpec((H, H), full),
            pl.BlockSpec((1, H), full), pl.BlockSpec((NC, BN, H), blk3),
        ],
        out_specs=[pl.BlockSpec((BN, H), blk), pl.BlockSpec((BN, H), blk),
                   pl.BlockSpec((BN, H), blk)],
        out_shape=[jax.ShapeDtypeStruct((N, H), F32),
                   jax.ShapeDtypeStruct((N, H), F32),
                   jax.ShapeDtypeStruct((N, H), F32)],
    )


def _k5_dec(N, H, D, NC, BN):
    ng = N // BN

    def body(n0, n1, sdacc1, wd1, bd1, wd2p, bd2p, wi1, bi1, wi2, bi2,
             o0, o1, p0, p1, sd1):
        h00 = jnp.maximum(_dot(n0[...], wd1[...]) + bd1[...], 0.0)
        h01 = jnp.maximum(_dot(n1[...], wd1[...]) + bd1[...], 0.0)
        o0[...] = _dot(h00, wd2p[...]) + bd2p[...]
        o1[...] = _dot(h01, wd2p[...]) + bd2p[...]
        h10 = jnp.maximum(_dot(n0[...], wi1[...]) + bi1[...], 0.0)
        h11 = jnp.maximum(_dot(n1[...], wi1[...]) + bi1[...], 0.0)
        p0[...] = _dot(h10, wi2[...]) + bi2[...]
        p1[...] = _dot(h11, wi2[...]) + bi2[...]
        sd1[...] = sdacc1[0] + sdacc1[1]

    full = lambda i: (0, 0)
    blk = lambda i: (i, 0)
    blk3 = lambda i: (0, i, 0)
    return pl.pallas_call(
        body,
        grid=(ng,),
        in_specs=[
            pl.BlockSpec((BN, H), blk), pl.BlockSpec((BN, H), blk),
            pl.BlockSpec((NC, BN, H), blk3),
            pl.BlockSpec((H, H), full), pl.BlockSpec((1, H), full),
            pl.BlockSpec((H, D), full), pl.BlockSpec((1, D), full),
            pl.BlockSpec((H, H), full), pl.BlockSpec((1, H), full),
            pl.BlockSpec((H, D), full), pl.BlockSpec((1, D), full),
        ],
        out_specs=[pl.BlockSpec((BN, D), blk), pl.BlockSpec((BN, D), blk),
                   pl.BlockSpec((BN, D), blk), pl.BlockSpec((BN, D), blk),
                   pl.BlockSpec((BN, H), blk)],
        out_shape=[jax.ShapeDtypeStruct((N, D), F32),
                   jax.ShapeDtypeStruct((N, D), F32),
                   jax.ShapeDtypeStruct((N, D), F32),
                   jax.ShapeDtypeStruct((N, D), F32),
                   jax.ShapeDtypeStruct((N, H), F32)],
    )


def kernel(x, edge_index, edge_attr, global_attr, sp_L_values, coeff,
           num_processing_steps, emb, Wenc, benc, Web, beb, Wnb, bnb, Wgb,
           bgb, Wd1, bd1, Wd2, bd2, Wi1, bi1, Wi2, bi2):
    T, N, D = x.shape
    E = edge_index.shape[1]
    H = Wenc.shape[1]
    K = emb.shape[0]
    OUT = Wd2.shape[1]
    assert T == 2

    info = plsc.get_sparse_core_info()
    NC = info.num_cores
    BN = 2000

    BE = 4000

    W_he, W_hxs, W_hxd, W_ie, W_ixs, W_ixd, W_g = (
        Web[i * H:(i + 1) * H] for i in range(7))
    Wn_hx, Wn_ix, Wn_recv, Wn_sent, Wn_g = (
        Wnb[i * H:(i + 1) * H] for i in range(5))

    g0 = global_attr
    r = lambda v: v.reshape(1, -1)
    zeros_nh = jnp.zeros((N, H), F32)
    coeff16 = jnp.concatenate([coeff, jnp.zeros((15,), F32)])
    Wd2p = jnp.pad(Wd2, ((0, 0), (0, D - OUT)))
    bd2p = jnp.pad(bd2, (0, D - OUT)).reshape(1, D)

    ix0, ix1, S0, D0, embW, table0, nbias0 = _k1_prep(N, D, H, K, BN)(
        x[0], x[1], Wenc, r(benc), W_ixs, W_ixd, emb, W_ie, W_g, r(beb),
        g0, Wn_g, r(bnb))

    e0, racc0, sacc0 = _edge_pass_a(E, N, H, with_hep=False,
                                    write_enew=True)(
        edge_index, edge_attr[0], S0, D0, table0, zeros_nh)

    n0, S1, D1, table1, nbias1, _, _ = _k2_node0(N, E, H, K, NC, BN)(
        ix0, ix1, racc0, sacc0, Wn_ix, Wn_recv, Wn_sent, nbias0,
        W_hxs, W_hxd, W_ixs, W_ixd, embW, W_g, r(beb), Wgb, r(bgb),
        Wn_g, r(bnb), g0)

    sdacc0 = _edge_pass_b(E, N, H)(edge_index, sp_L_values, coeff16, n0,
                                   zeros_nh)

    hep1 = _k3_heproj(E, H, BE)(e0, W_he)

    racc1, sacc1 = _edge_pass_a(E, N, H, with_hep=True,
                                write_enew=False)(
        edge_index, edge_attr[1], S1, D1, table1, hep1, zeros_nh)

    n1, td1, sd0 = _k4_node1(N, H, NC, BN)(
        n0, ix1, racc1, sacc1, Wn_hx, Wn_ix, Wn_recv, Wn_sent, nbias1,
        sdacc0)

    sdacc1 = _edge_pass_b(E, N, H)(edge_index, sp_L_values, coeff16, n1,
                                   zeros_nh)

    o0, o1, p0, p1, sd1 = _k5_dec(N, H, D, NC, BN)(
        n0, n1, sdacc1, Wd1, r(bd1), Wd2p, bd2p, Wi1, r(bi1), Wi2, r(bi2))

    out_nodes = jnp.stack([o0[:, :OUT], o1[:, :OUT]])
    time_derivatives = jnp.stack([n0, td1])
    spatial_derivatives = jnp.stack([sd0, sd1])
    pred_inputs = jnp.stack([p0, p1])
    return (out_nodes, time_derivatives, spatial_derivatives, pred_inputs)
